# Initial kernel scaffold; baseline (speedup 1.0000x reference)
import jax
import jax.numpy as jnp
from jax.experimental import pallas as pl


def kernel(x, sub_edge_index, node_to_subgraph, W1, b1, gamma, beta, W2, b2):
    raise NotImplementedError("write your pallas kernel here")


# speedup vs baseline: 9.3467x; 9.3467x over previous
import functools

import jax
import jax.numpy as jnp
from jax import lax
from jax.experimental import pallas as pl
from jax.experimental.pallas import tpu as pltpu
from jax.experimental.pallas import tpu_sc as plsc

N = 10000
E = 320000
D = 128
S = 512

NC = 2
NS = 16
NW = NC * NS
ET = E // NW
EK = 80
NCHUNK = ET // EK
NPAD = 10240
RPT = NPAD // NS
RB = 400
GRID = N // RB
PK = 80
PCH = N // PK
SPT = S // NS


def _mesh():
    return plsc.VectorSubcoreMesh(
        core_axis_name="c", subcore_axis_name="s", num_cores=NC, num_subcores=NS)


NPR = NPAD // 128


@functools.cache
def _make_deg_kernel():
    return functools.partial(
        pl.kernel,
        out_type=jax.ShapeDtypeStruct((NC, NPR, 128), jnp.float32),
        mesh=_mesh(),
        scratch_types=[
            pltpu.VMEM((EK,), jnp.int32),
            pltpu.VMEM((EK,), jnp.int32),
            pltpu.VMEM((EK,), jnp.int32),
            pltpu.VMEM((EK, 128), jnp.float32),
            pltpu.VMEM_SHARED((NPR, 128), jnp.float32),
            pltpu.SemaphoreType.DMA,
        ],
    )(_deg_body)


def _deg_body(dst_hbm, eye_hbm, zero8_hbm, out_hbm,
              idx_v, r_v, c_v, oneh_v, acc, sem):
    c = lax.axis_index("c")
    s = lax.axis_index("s")
    t = c * NS + s

    @pl.when(s < NPR // 8)
    def _():
        pltpu.sync_copy(zero8_hbm, acc.at[pl.ds(s * 8, 8)])

    plsc.subcore_barrier()

    def body(j, carry):
        base = pl.multiple_of(t * ET + j * EK, 8)
        pltpu.sync_copy(dst_hbm.at[pl.ds(base, EK)], idx_v)
        for k in range(EK // 16):
            idx16 = idx_v[pl.ds(16 * k, 16)]
            r_v[pl.ds(16 * k, 16)] = lax.shift_right_logical(idx16, 7)
            c_v[pl.ds(16 * k, 16)] = lax.bitwise_and(idx16, 127)
        pltpu.async_copy(eye_hbm.at[c_v], oneh_v, sem).wait()
        pltpu.sync_copy(oneh_v, acc.at[r_v], add=True)
        return carry

    lax.fori_loop(0, NCHUNK, body, 0)
    plsc.subcore_barrier()

    @pl.when(s < NPR // 8)
    def _():
        pltpu.sync_copy(acc.at[pl.ds(s * 8, 8)], out_hbm.at[c, pl.ds(s * 8, 8)])


@functools.cache
def _make_agg_kernel():
    return functools.partial(
        pl.kernel,
        out_type=jax.ShapeDtypeStruct((NC, NPAD, D), jnp.float32),
        mesh=_mesh(),
        scratch_types=[
            pltpu.VMEM((EK,), jnp.int32),
            pltpu.VMEM((EK,), jnp.int32),
            pltpu.VMEM((EK, D), jnp.float32),
            pltpu.VMEM_SHARED((NPAD, D), jnp.float32),
            pltpu.SemaphoreType.DMA,
        ],
    )(_agg_body)


def _agg_body(hs_hbm, src_hbm, dst_hbm, zeros_hbm, out_hbm,
              src_v, dst_v, rows_v, acc, sem):
    c = lax.axis_index("c")
    s = lax.axis_index("s")
    t = c * NS + s
    pltpu.sync_copy(zeros_hbm, acc.at[pl.ds(s * RPT, RPT)])
    plsc.subcore_barrier()

    def body(j, carry):
        base = pl.multiple_of(t * ET + j * EK, 8)
        pltpu.sync_copy(src_hbm.at[pl.ds(base, EK)], src_v)
        pltpu.sync_copy(dst_hbm.at[pl.ds(base, EK)], dst_v)
        pltpu.async_copy(hs_hbm.at[src_v], rows_v, sem).wait()
        pltpu.sync_copy(rows_v, acc.at[dst_v], add=True)
        return carry

    lax.fori_loop(0, NCHUNK, body, 0)
    plsc.subcore_barrier()
    pltpu.sync_copy(acc.at[pl.ds(s * RPT, RPT)],
                    out_hbm.at[c, pl.ds(s * RPT, RPT)])


@functools.cache
def _make_pool_kernel():
    return functools.partial(
        pl.kernel,
        out_type=[
            jax.ShapeDtypeStruct((NC, S, D), jnp.float32),
            jax.ShapeDtypeStruct((NC, S, D), jnp.float32),
            jax.ShapeDtypeStruct((NC, 16, 128), jnp.float32),
        ],
        mesh=_mesh(),
        scratch_types=[
            pltpu.VMEM((PK,), jnp.int32),
            pltpu.VMEM((PK, D), jnp.float32),
            pltpu.VMEM((PK, D), jnp.float32),
            pltpu.VMEM((PK,), jnp.int32),
            pltpu.VMEM((PK,), jnp.int32),
            pltpu.VMEM((PK, 128), jnp.float32),
            pltpu.VMEM_SHARED((S, D), jnp.float32),
            pltpu.VMEM_SHARED((S, D), jnp.float32),
            pltpu.VMEM_SHARED((16, 128), jnp.float32),
            pltpu.SemaphoreType.DMA,
        ],
    )(_pool_body)


def _pool_body(h1_hbm, h2_hbm, seg_hbm, zrow_hbm, eye_hbm,
               out1_hbm, out2_hbm, outc_hbm,
               seg_v, r1_v, r2_v, rr_v, cc_v, oneh_v, acc1, acc2, accc, sem):
    c = lax.axis_index("c")
    s = lax.axis_index("s")
    t = c * NS + s
    pltpu.sync_copy(zrow_hbm, acc1.at[pl.ds(s * SPT, SPT)])
    pltpu.sync_copy(zrow_hbm, acc2.at[pl.ds(s * SPT, SPT)])

    @pl.when(s < 2)
    def _():
        pltpu.sync_copy(zrow_hbm.at[pl.ds(0, 8)], accc.at[pl.ds(s * 8, 8)])

    plsc.subcore_barrier()

    for k in range(4):
        cid = k * NW + t

        @pl.when(cid < PCH)
        def _():
            base = pl.multiple_of(cid * PK, 8)
            pltpu.sync_copy(seg_hbm.at[pl.ds(base, PK)], seg_v)
            pltpu.sync_copy(h1_hbm.at[pl.ds(base, PK)], r1_v)
            pltpu.sync_copy(h2_hbm.at[pl.ds(base, PK)], r2_v)
            pltpu.sync_copy(r1_v, acc1.at[seg_v], add=True)
            pltpu.sync_copy(r2_v, acc2.at[seg_v], add=True)
            for q in range(PK // 16):
                seg16 = seg_v[pl.ds(16 * q, 16)]
                rr_v[pl.ds(16 * q, 16)] = lax.shift_right_logical(seg16, 7)
                cc_v[pl.ds(16 * q, 16)] = lax.bitwise_and(seg16, 127)
            pltpu.async_copy(eye_hbm.at[cc_v], oneh_v, sem).wait()
            pltpu.sync_copy(oneh_v, accc.at[rr_v], add=True)

    plsc.subcore_barrier()
    pltpu.sync_copy(acc1.at[pl.ds(s * SPT, SPT)], out1_hbm.at[c, pl.ds(s * SPT, SPT)])
    pltpu.sync_copy(acc2.at[pl.ds(s * SPT, SPT)], out2_hbm.at[c, pl.ds(s * SPT, SPT)])

    @pl.when(s < 2)
    def _():
        pltpu.sync_copy(accc.at[pl.ds(s * 8, 8)], outc_hbm.at[c, pl.ds(s * 8, 8)])


def _mm1_body(x_ref, w_ref, d0_ref, d1_ref, hs_ref, dis_ref):
    deg = d0_ref[0] + d1_ref[0] + 1.0
    dis = lax.rsqrt(deg)
    m = jnp.dot(x_ref[...], w_ref[...], preferred_element_type=jnp.float32)
    hs_ref[...] = m * dis
    dis_ref[...] = jnp.broadcast_to(dis, (RB, 16))


def _ep1_body(p0_ref, p1_ref, hs_ref, dis_ref, b_ref, g_ref, stats_ref):
    g = dis_ref[:, 0:1] * (p0_ref[0] + p1_ref[0] + hs_ref[...]) + b_ref[...]
    g_ref[...] = g

    @pl.when(pl.program_id(0) == 0)
    def _():
        stats_ref[...] = jnp.zeros_like(stats_ref)

    upd = jnp.concatenate(
        [jnp.sum(g, axis=0, keepdims=True),
         jnp.sum(g * g, axis=0, keepdims=True),
         jnp.zeros((6, D), jnp.float32)], axis=0)
    stats_ref[...] += upd


def _bn_mm2_body(g_ref, stats_ref, gamma_ref, beta_ref, w_ref, dis_ref,
                 h1_ref, hs2_ref):
    inv_n = 1.0 / N
    mean = stats_ref[0:1, :] * inv_n
    var = stats_ref[1:2, :] * inv_n - mean * mean
    inv = lax.rsqrt(var + 1e-5)
    h1 = (g_ref[...] - mean) * inv * gamma_ref[...] + beta_ref[...]
    h1 = jnp.maximum(h1, 0.0)
    h1_ref[...] = h1
    m = jnp.dot(h1, w_ref[...], preferred_element_type=jnp.float32)
    hs2_ref[...] = m * dis_ref[:, 0:1]


def _ep2_body(p0_ref, p1_ref, hs_ref, dis_ref, b_ref, h2_ref):
    g = dis_ref[:, 0:1] * (p0_ref[0] + p1_ref[0] + hs_ref[...]) + b_ref[...]
    h2_ref[...] = jnp.maximum(g, 0.0)


def _final_body(s1_ref, s2_ref, c0_ref, c1_ref, out_ref):
    cnt = jnp.maximum(c0_ref[0] + c1_ref[0], 1.0)
    out_ref[:, :D] = (s1_ref[0] + s1_ref[1]) / cnt
    out_ref[:, D:] = (s2_ref[0] + s2_ref[1]) / cnt


def _row_spec(shape):
    return pl.BlockSpec(shape, lambda i: (i, 0))


def _fixed_spec(shape):
    return pl.BlockSpec(shape, lambda i: tuple(0 for _ in shape))


def _part_spec(core):
    return pl.BlockSpec((1, RB, D), lambda i, c=core: (c, i, 0))


def kernel(x, sub_edge_index, node_to_subgraph, W1, b1, gamma, beta, W2, b2):
    src = sub_edge_index[0]
    dst = sub_edge_index[1]
    zrows = jnp.zeros((RPT, D), jnp.float32)
    eye = jnp.eye(128, dtype=jnp.float32)

    degp = _make_deg_kernel()(dst, eye,
                              jnp.zeros((8, 128), jnp.float32)).reshape(NC, NPAD, 1)

    hs1, dis = pl.pallas_call(
        _mm1_body,
        grid=(GRID,),
        in_specs=[_row_spec((RB, D)), _fixed_spec((D, D)),
                  pl.BlockSpec((1, RB, 1), lambda i: (0, i, 0)),
                  pl.BlockSpec((1, RB, 1), lambda i: (1, i, 0))],
        out_specs=[_row_spec((RB, D)), _row_spec((RB, 16))],
        out_shape=[jax.ShapeDtypeStruct((N, D), jnp.float32),
                   jax.ShapeDtypeStruct((N, 16), jnp.float32)],
    )(x, W1, degp, degp)

    parts1 = _make_agg_kernel()(hs1, src, dst, zrows)

    g1, stats = pl.pallas_call(
        _ep1_body,
        grid=(GRID,),
        in_specs=[_part_spec(0), _part_spec(1), _row_spec((RB, D)),
                  _row_spec((RB, 16)), _fixed_spec((1, D))],
        out_specs=[_row_spec((RB, D)), _fixed_spec((8, D))],
        out_shape=[jax.ShapeDtypeStruct((N, D), jnp.float32),
                   jax.ShapeDtypeStruct((8, D), jnp.float32)],
    )(parts1, parts1, hs1, dis, b1.reshape(1, D))

    h1, hs2 = pl.pallas_call(
        _bn_mm2_body,
        grid=(GRID,),
        in_specs=[_row_spec((RB, D)), _fixed_spec((8, D)), _fixed_spec((1, D)),
                  _fixed_spec((1, D)), _fixed_spec((D, D)), _row_spec((RB, 16))],
        out_specs=[_row_spec((RB, D)), _row_spec((RB, D))],
        out_shape=[jax.ShapeDtypeStruct((N, D), jnp.float32),
                   jax.ShapeDtypeStruct((N, D), jnp.float32)],
    )(g1, stats, gamma.reshape(1, D), beta.reshape(1, D), W2, dis)

    parts2 = _make_agg_kernel()(hs2, src, dst, zrows)

    h2 = pl.pallas_call(
        _ep2_body,
        grid=(GRID,),
        in_specs=[_part_spec(0), _part_spec(1), _row_spec((RB, D)),
                  _row_spec((RB, 16)), _fixed_spec((1, D))],
        out_specs=_row_spec((RB, D)),
        out_shape=jax.ShapeDtypeStruct((N, D), jnp.float32),
    )(parts2, parts2, hs2, dis, b2.reshape(1, D))

    psum1, psum2, pcnt = _make_pool_kernel()(
        h1, h2, node_to_subgraph, jnp.zeros((SPT, D), jnp.float32), eye)
    pcnt = pcnt.reshape(NC, 16 * 128, 1)

    out = pl.pallas_call(
        _final_body,
        grid=(1,),
        in_specs=[pl.BlockSpec((NC, S, D), lambda i: (0, 0, 0)),
                  pl.BlockSpec((NC, S, D), lambda i: (0, 0, 0)),
                  pl.BlockSpec((1, S, 1), lambda i: (0, 0, 0)),
                  pl.BlockSpec((1, S, 1), lambda i: (1, 0, 0))],
        out_specs=pl.BlockSpec((S, 2 * D), lambda i: (0, 0)),
        out_shape=jax.ShapeDtypeStruct((S, 2 * D), jnp.float32),
    )(psum1, psum2, pcnt, pcnt)
    return out


# speedup vs baseline: 12.2284x; 1.3083x over previous
import functools

import jax
import jax.numpy as jnp
from jax import lax
from jax.experimental import pallas as pl
from jax.experimental.pallas import tpu as pltpu
from jax.experimental.pallas import tpu_sc as plsc

N = 10000
E = 320000
D = 128
S = 512

NC = 2
NS = 16
NW = NC * NS
ET = E // NW
EK = 80
NCHUNK = ET // EK
NPAD = 10240
RPT = NPAD // NS
RB = 400
GRID = N // RB
PK = 80
PCH = N // PK
SPT = S // NS


def _mesh():
    return plsc.VectorSubcoreMesh(
        core_axis_name="c", subcore_axis_name="s", num_cores=NC, num_subcores=NS)


NPR = NPAD // 128


@functools.cache
def _make_deg_kernel():
    return functools.partial(
        pl.kernel,
        out_type=jax.ShapeDtypeStruct((NC, NPR, 128), jnp.float32),
        mesh=_mesh(),
        scratch_types=[
            pltpu.VMEM((EK,), jnp.int32),
            pltpu.VMEM((EK,), jnp.int32),
            pltpu.VMEM((EK,), jnp.int32),
            pltpu.VMEM((EK, 128), jnp.float32),
            pltpu.VMEM_SHARED((NPR, 128), jnp.float32),
            pltpu.SemaphoreType.DMA,
        ],
    )(_deg_body)


def _deg_body(dst_hbm, eye_hbm, zero8_hbm, out_hbm,
              idx_v, r_v, c_v, oneh_v, acc, sem):
    c = lax.axis_index("c")
    s = lax.axis_index("s")
    t = c * NS + s

    @pl.when(s < NPR // 8)
    def _():
        pltpu.sync_copy(zero8_hbm, acc.at[pl.ds(s * 8, 8)])

    plsc.subcore_barrier()

    def body(j, carry):
        base = pl.multiple_of(t * ET + j * EK, 8)
        pltpu.sync_copy(dst_hbm.at[pl.ds(base, EK)], idx_v)
        for k in range(EK // 16):
            idx16 = idx_v[pl.ds(16 * k, 16)]
            r_v[pl.ds(16 * k, 16)] = lax.shift_right_logical(idx16, 7)
            c_v[pl.ds(16 * k, 16)] = lax.bitwise_and(idx16, 127)
        pltpu.async_copy(eye_hbm.at[c_v], oneh_v, sem).wait()
        pltpu.sync_copy(oneh_v, acc.at[r_v], add=True)
        return carry

    lax.fori_loop(0, NCHUNK, body, 0)
    plsc.subcore_barrier()

    @pl.when(s < NPR // 8)
    def _():
        pltpu.sync_copy(acc.at[pl.ds(s * 8, 8)], out_hbm.at[c, pl.ds(s * 8, 8)])


@functools.cache
def _make_agg_kernel():
    return functools.partial(
        pl.kernel,
        out_type=jax.ShapeDtypeStruct((NC, NPAD, D), jnp.float32),
        mesh=_mesh(),
        scratch_types=[
            pltpu.VMEM((EK,), jnp.int32),
            pltpu.VMEM((EK,), jnp.int32),
            pltpu.VMEM((EK,), jnp.int32),
            pltpu.VMEM((EK,), jnp.int32),
            pltpu.VMEM((EK, D), jnp.float32),
            pltpu.VMEM((EK, D), jnp.float32),
            pltpu.VMEM_SHARED((NPAD, D), jnp.float32),
            pltpu.SemaphoreType.DMA,
            pltpu.SemaphoreType.DMA,
        ],
    )(_agg_body)


def _agg_body(hs_hbm, src_hbm, dst_hbm, zeros_hbm, out_hbm,
              src0_v, dst0_v, src1_v, dst1_v, rows0_v, rows1_v,
              acc, sem0, sem1):
    c = lax.axis_index("c")
    s = lax.axis_index("s")
    t = c * NS + s
    pltpu.sync_copy(zeros_hbm, acc.at[pl.ds(s * RPT, RPT)])
    plsc.subcore_barrier()

    def load_and_gather(j, idx_s, idx_d, rows, sem):
        base = pl.multiple_of(t * ET + j * EK, 8)
        pltpu.sync_copy(src_hbm.at[pl.ds(base, EK)], idx_s)
        pltpu.sync_copy(dst_hbm.at[pl.ds(base, EK)], idx_d)
        pltpu.async_copy(hs_hbm.at[idx_s], rows, sem)

    load_and_gather(0, src0_v, dst0_v, rows0_v, sem0)

    def body(i, carry):
        j0 = 2 * i
        load_and_gather(j0 + 1, src1_v, dst1_v, rows1_v, sem1)
        pltpu.make_async_copy(hs_hbm.at[src0_v], rows0_v, sem0).wait()
        pltpu.sync_copy(rows0_v, acc.at[dst0_v], add=True)
        load_and_gather(j0 + 2, src0_v, dst0_v, rows0_v, sem0)
        pltpu.make_async_copy(hs_hbm.at[src1_v], rows1_v, sem1).wait()
        pltpu.sync_copy(rows1_v, acc.at[dst1_v], add=True)
        return carry

    lax.fori_loop(0, (NCHUNK - 1) // 2, body, 0)
    pltpu.make_async_copy(hs_hbm.at[src0_v], rows0_v, sem0).wait()
    pltpu.sync_copy(rows0_v, acc.at[dst0_v], add=True)
    plsc.subcore_barrier()
    pltpu.sync_copy(acc.at[pl.ds(s * RPT, RPT)],
                    out_hbm.at[c, pl.ds(s * RPT, RPT)])


@functools.cache
def _make_pool_kernel():
    return functools.partial(
        pl.kernel,
        out_type=[
            jax.ShapeDtypeStruct((NC, S, D), jnp.float32),
            jax.ShapeDtypeStruct((NC, S, D), jnp.float32),
            jax.ShapeDtypeStruct((NC, 16, 128), jnp.float32),
        ],
        mesh=_mesh(),
        scratch_types=[
            pltpu.VMEM((PK,), jnp.int32),
            pltpu.VMEM((PK, D), jnp.float32),
            pltpu.VMEM((PK, D), jnp.float32),
            pltpu.VMEM((PK,), jnp.int32),
            pltpu.VMEM((PK,), jnp.int32),
            pltpu.VMEM((PK, 128), jnp.float32),
            pltpu.VMEM_SHARED((S, D), jnp.float32),
            pltpu.VMEM_SHARED((S, D), jnp.float32),
            pltpu.VMEM_SHARED((16, 128), jnp.float32),
            pltpu.SemaphoreType.DMA,
        ],
    )(_pool_body)


def _pool_body(h1_hbm, h2_hbm, seg_hbm, zrow_hbm, eye_hbm,
               out1_hbm, out2_hbm, outc_hbm,
               seg_v, r1_v, r2_v, rr_v, cc_v, oneh_v, acc1, acc2, accc, sem):
    c = lax.axis_index("c")
    s = lax.axis_index("s")
    t = c * NS + s
    pltpu.sync_copy(zrow_hbm, acc1.at[pl.ds(s * SPT, SPT)])
    pltpu.sync_copy(zrow_hbm, acc2.at[pl.ds(s * SPT, SPT)])

    @pl.when(s < 2)
    def _():
        pltpu.sync_copy(zrow_hbm.at[pl.ds(0, 8)], accc.at[pl.ds(s * 8, 8)])

    plsc.subcore_barrier()

    for k in range(4):
        cid = k * NW + t

        @pl.when(cid < PCH)
        def _():
            base = pl.multiple_of(cid * PK, 8)
            pltpu.sync_copy(seg_hbm.at[pl.ds(base, PK)], seg_v)
            pltpu.sync_copy(h1_hbm.at[pl.ds(base, PK)], r1_v)
            pltpu.sync_copy(h2_hbm.at[pl.ds(base, PK)], r2_v)
            pltpu.sync_copy(r1_v, acc1.at[seg_v], add=True)
            pltpu.sync_copy(r2_v, acc2.at[seg_v], add=True)
            for q in range(PK // 16):
                seg16 = seg_v[pl.ds(16 * q, 16)]
                rr_v[pl.ds(16 * q, 16)] = lax.shift_right_logical(seg16, 7)
                cc_v[pl.ds(16 * q, 16)] = lax.bitwise_and(seg16, 127)
            pltpu.async_copy(eye_hbm.at[cc_v], oneh_v, sem).wait()
            pltpu.sync_copy(oneh_v, accc.at[rr_v], add=True)

    plsc.subcore_barrier()
    pltpu.sync_copy(acc1.at[pl.ds(s * SPT, SPT)], out1_hbm.at[c, pl.ds(s * SPT, SPT)])
    pltpu.sync_copy(acc2.at[pl.ds(s * SPT, SPT)], out2_hbm.at[c, pl.ds(s * SPT, SPT)])

    @pl.when(s < 2)
    def _():
        pltpu.sync_copy(accc.at[pl.ds(s * 8, 8)], outc_hbm.at[c, pl.ds(s * 8, 8)])


def _mm1_body(x_ref, w_ref, d0_ref, d1_ref, hs_ref, dis_ref):
    deg = d0_ref[0] + d1_ref[0] + 1.0
    dis = lax.rsqrt(deg)
    m = jnp.dot(x_ref[...], w_ref[...], preferred_element_type=jnp.float32)
    hs_ref[...] = m * dis
    dis_ref[...] = jnp.broadcast_to(dis, (RB, 16))


def _ep1_body(p0_ref, p1_ref, hs_ref, dis_ref, b_ref, g_ref, stats_ref):
    g = dis_ref[:, 0:1] * (p0_ref[0] + p1_ref[0] + hs_ref[...]) + b_ref[...]
    g_ref[...] = g

    @pl.when(pl.program_id(0) == 0)
    def _():
        stats_ref[...] = jnp.zeros_like(stats_ref)

    upd = jnp.concatenate(
        [jnp.sum(g, axis=0, keepdims=True),
         jnp.sum(g * g, axis=0, keepdims=True),
         jnp.zeros((6, D), jnp.float32)], axis=0)
    stats_ref[...] += upd


def _bn_mm2_body(g_ref, stats_ref, gamma_ref, beta_ref, w_ref, dis_ref,
                 h1_ref, hs2_ref):
    inv_n = 1.0 / N
    mean = stats_ref[0:1, :] * inv_n
    var = stats_ref[1:2, :] * inv_n - mean * mean
    inv = lax.rsqrt(var + 1e-5)
    h1 = (g_ref[...] - mean) * inv * gamma_ref[...] + beta_ref[...]
    h1 = jnp.maximum(h1, 0.0)
    h1_ref[...] = h1
    m = jnp.dot(h1, w_ref[...], preferred_element_type=jnp.float32)
    hs2_ref[...] = m * dis_ref[:, 0:1]


def _ep2_body(p0_ref, p1_ref, hs_ref, dis_ref, b_ref, h2_ref):
    g = dis_ref[:, 0:1] * (p0_ref[0] + p1_ref[0] + hs_ref[...]) + b_ref[...]
    h2_ref[...] = jnp.maximum(g, 0.0)


def _final_body(s1_ref, s2_ref, c0_ref, c1_ref, out_ref):
    cnt = jnp.maximum(c0_ref[0] + c1_ref[0], 1.0)
    out_ref[:, :D] = (s1_ref[0] + s1_ref[1]) / cnt
    out_ref[:, D:] = (s2_ref[0] + s2_ref[1]) / cnt


def _row_spec(shape):
    return pl.BlockSpec(shape, lambda i: (i, 0))


def _fixed_spec(shape):
    return pl.BlockSpec(shape, lambda i: tuple(0 for _ in shape))


def _part_spec(core):
    return pl.BlockSpec((1, RB, D), lambda i, c=core: (c, i, 0))


def kernel(x, sub_edge_index, node_to_subgraph, W1, b1, gamma, beta, W2, b2):
    src = sub_edge_index[0]
    dst = sub_edge_index[1]
    zrows = jnp.zeros((RPT, D), jnp.float32)
    eye = jnp.eye(128, dtype=jnp.float32)

    degp = _make_deg_kernel()(dst, eye,
                              jnp.zeros((8, 128), jnp.float32)).reshape(NC, NPAD, 1)

    hs1, dis = pl.pallas_call(
        _mm1_body,
        grid=(GRID,),
        in_specs=[_row_spec((RB, D)), _fixed_spec((D, D)),
                  pl.BlockSpec((1, RB, 1), lambda i: (0, i, 0)),
                  pl.BlockSpec((1, RB, 1), lambda i: (1, i, 0))],
        out_specs=[_row_spec((RB, D)), _row_spec((RB, 16))],
        out_shape=[jax.ShapeDtypeStruct((N, D), jnp.float32),
                   jax.ShapeDtypeStruct((N, 16), jnp.float32)],
    )(x, W1, degp, degp)

    parts1 = _make_agg_kernel()(hs1, src, dst, zrows)

    g1, stats = pl.pallas_call(
        _ep1_body,
        grid=(GRID,),
        in_specs=[_part_spec(0), _part_spec(1), _row_spec((RB, D)),
                  _row_spec((RB, 16)), _fixed_spec((1, D))],
        out_specs=[_row_spec((RB, D)), _fixed_spec((8, D))],
        out_shape=[jax.ShapeDtypeStruct((N, D), jnp.float32),
                   jax.ShapeDtypeStruct((8, D), jnp.float32)],
    )(parts1, parts1, hs1, dis, b1.reshape(1, D))

    h1, hs2 = pl.pallas_call(
        _bn_mm2_body,
        grid=(GRID,),
        in_specs=[_row_spec((RB, D)), _fixed_spec((8, D)), _fixed_spec((1, D)),
                  _fixed_spec((1, D)), _fixed_spec((D, D)), _row_spec((RB, 16))],
        out_specs=[_row_spec((RB, D)), _row_spec((RB, D))],
        out_shape=[jax.ShapeDtypeStruct((N, D), jnp.float32),
                   jax.ShapeDtypeStruct((N, D), jnp.float32)],
    )(g1, stats, gamma.reshape(1, D), beta.reshape(1, D), W2, dis)

    parts2 = _make_agg_kernel()(hs2, src, dst, zrows)

    h2 = pl.pallas_call(
        _ep2_body,
        grid=(GRID,),
        in_specs=[_part_spec(0), _part_spec(1), _row_spec((RB, D)),
                  _row_spec((RB, 16)), _fixed_spec((1, D))],
        out_specs=_row_spec((RB, D)),
        out_shape=jax.ShapeDtypeStruct((N, D), jnp.float32),
    )(parts2, parts2, hs2, dis, b2.reshape(1, D))

    psum1, psum2, pcnt = _make_pool_kernel()(
        h1, h2, node_to_subgraph, jnp.zeros((SPT, D), jnp.float32), eye)
    pcnt = pcnt.reshape(NC, 16 * 128, 1)

    out = pl.pallas_call(
        _final_body,
        grid=(1,),
        in_specs=[pl.BlockSpec((NC, S, D), lambda i: (0, 0, 0)),
                  pl.BlockSpec((NC, S, D), lambda i: (0, 0, 0)),
                  pl.BlockSpec((1, S, 1), lambda i: (0, 0, 0)),
                  pl.BlockSpec((1, S, 1), lambda i: (1, 0, 0))],
        out_specs=pl.BlockSpec((S, 2 * D), lambda i: (0, 0)),
        out_shape=jax.ShapeDtypeStruct((S, 2 * D), jnp.float32),
    )(psum1, psum2, pcnt, pcnt)
    return out


# speedup vs baseline: 12.3348x; 1.0087x over previous
import functools

import jax
import jax.numpy as jnp
from jax import lax
from jax.experimental import pallas as pl
from jax.experimental.pallas import tpu as pltpu
from jax.experimental.pallas import tpu_sc as plsc

N = 10000
E = 320000
D = 128
S = 512

NC = 2
NS = 16
NW = NC * NS
ET = E // NW
EK = 80
NCHUNK = ET // EK
NPAD = 10240
RPT = NPAD // NS
RB = 400
GRID = N // RB
PK = 80
PCH = N // PK
SPT = S // NS


def _mesh():
    return plsc.VectorSubcoreMesh(
        core_axis_name="c", subcore_axis_name="s", num_cores=NC, num_subcores=NS)


NPR = NPAD // 128


@functools.cache
def _make_deg_kernel():
    return functools.partial(
        pl.kernel,
        out_type=jax.ShapeDtypeStruct((NC, NPR, 128), jnp.float32),
        mesh=_mesh(),
        scratch_types=[
            pltpu.VMEM((EK,), jnp.int32),
            pltpu.VMEM((EK,), jnp.int32),
            pltpu.VMEM((EK,), jnp.int32),
            pltpu.VMEM((EK,), jnp.int32),
            pltpu.VMEM((EK,), jnp.int32),
            pltpu.VMEM((EK,), jnp.int32),
            pltpu.VMEM((EK, 128), jnp.float32),
            pltpu.VMEM((EK, 128), jnp.float32),
            pltpu.VMEM_SHARED((NPR, 128), jnp.float32),
            pltpu.SemaphoreType.DMA,
            pltpu.SemaphoreType.DMA,
        ],
    )(_deg_body)


def _deg_body(dst_hbm, eye_hbm, zero8_hbm, out_hbm,
              idx0_v, r0_v, c0_v, idx1_v, r1_v, c1_v, oneh0_v, oneh1_v,
              acc, sem0, sem1):
    c = lax.axis_index("c")
    s = lax.axis_index("s")
    t = c * NS + s

    @pl.when(s < NPR // 8)
    def _():
        pltpu.sync_copy(zero8_hbm, acc.at[pl.ds(s * 8, 8)])

    plsc.subcore_barrier()

    def load_and_gather(j, idx_v, r_v, c_v, oneh_v, sem):
        base = pl.multiple_of(t * ET + j * EK, 8)
        pltpu.sync_copy(dst_hbm.at[pl.ds(base, EK)], idx_v)
        for k in range(EK // 16):
            idx16 = idx_v[pl.ds(16 * k, 16)]
            r_v[pl.ds(16 * k, 16)] = lax.shift_right_logical(idx16, 7)
            c_v[pl.ds(16 * k, 16)] = lax.bitwise_and(idx16, 127)
        pltpu.async_copy(eye_hbm.at[c_v], oneh_v, sem)

    load_and_gather(0, idx0_v, r0_v, c0_v, oneh0_v, sem0)

    def body(i, carry):
        j0 = 2 * i
        load_and_gather(j0 + 1, idx1_v, r1_v, c1_v, oneh1_v, sem1)
        pltpu.make_async_copy(eye_hbm.at[c0_v], oneh0_v, sem0).wait()
        pltpu.sync_copy(oneh0_v, acc.at[r0_v], add=True)
        load_and_gather(j0 + 2, idx0_v, r0_v, c0_v, oneh0_v, sem0)
        pltpu.make_async_copy(eye_hbm.at[c1_v], oneh1_v, sem1).wait()
        pltpu.sync_copy(oneh1_v, acc.at[r1_v], add=True)
        return carry

    lax.fori_loop(0, (NCHUNK - 1) // 2, body, 0)
    pltpu.make_async_copy(eye_hbm.at[c0_v], oneh0_v, sem0).wait()
    pltpu.sync_copy(oneh0_v, acc.at[r0_v], add=True)
    plsc.subcore_barrier()

    @pl.when(s < NPR // 8)
    def _():
        pltpu.sync_copy(acc.at[pl.ds(s * 8, 8)], out_hbm.at[c, pl.ds(s * 8, 8)])


@functools.cache
def _make_agg_kernel():
    return functools.partial(
        pl.kernel,
        out_type=jax.ShapeDtypeStruct((NC, NPAD, D), jnp.float32),
        mesh=_mesh(),
        scratch_types=[
            pltpu.VMEM((EK,), jnp.int32),
            pltpu.VMEM((EK,), jnp.int32),
            pltpu.VMEM((EK,), jnp.int32),
            pltpu.VMEM((EK,), jnp.int32),
            pltpu.VMEM((EK, D), jnp.float32),
            pltpu.VMEM((EK, D), jnp.float32),
            pltpu.VMEM_SHARED((NPAD, D), jnp.float32),
            pltpu.SemaphoreType.DMA,
            pltpu.SemaphoreType.DMA,
        ],
    )(_agg_body)


def _agg_body(hs_hbm, src_hbm, dst_hbm, zeros_hbm, out_hbm,
              src0_v, dst0_v, src1_v, dst1_v, rows0_v, rows1_v,
              acc, sem0, sem1):
    c = lax.axis_index("c")
    s = lax.axis_index("s")
    t = c * NS + s
    pltpu.sync_copy(zeros_hbm, acc.at[pl.ds(s * RPT, RPT)])
    plsc.subcore_barrier()

    def load_and_gather(j, idx_s, idx_d, rows, sem):
        base = pl.multiple_of(t * ET + j * EK, 8)
        pltpu.sync_copy(src_hbm.at[pl.ds(base, EK)], idx_s)
        pltpu.sync_copy(dst_hbm.at[pl.ds(base, EK)], idx_d)
        pltpu.async_copy(hs_hbm.at[idx_s], rows, sem)

    load_and_gather(0, src0_v, dst0_v, rows0_v, sem0)

    def body(i, carry):
        j0 = 2 * i
        load_and_gather(j0 + 1, src1_v, dst1_v, rows1_v, sem1)
        pltpu.make_async_copy(hs_hbm.at[src0_v], rows0_v, sem0).wait()
        pltpu.sync_copy(rows0_v, acc.at[dst0_v], add=True)
        load_and_gather(j0 + 2, src0_v, dst0_v, rows0_v, sem0)
        pltpu.make_async_copy(hs_hbm.at[src1_v], rows1_v, sem1).wait()
        pltpu.sync_copy(rows1_v, acc.at[dst1_v], add=True)
        return carry

    lax.fori_loop(0, (NCHUNK - 1) // 2, body, 0)
    pltpu.make_async_copy(hs_hbm.at[src0_v], rows0_v, sem0).wait()
    pltpu.sync_copy(rows0_v, acc.at[dst0_v], add=True)
    plsc.subcore_barrier()
    pltpu.sync_copy(acc.at[pl.ds(s * RPT, RPT)],
                    out_hbm.at[c, pl.ds(s * RPT, RPT)])


@functools.cache
def _make_pool_kernel():
    return functools.partial(
        pl.kernel,
        out_type=[
            jax.ShapeDtypeStruct((NC, S, D), jnp.float32),
            jax.ShapeDtypeStruct((NC, S, D), jnp.float32),
            jax.ShapeDtypeStruct((NC, 16, 128), jnp.float32),
        ],
        mesh=_mesh(),
        scratch_types=[
            pltpu.VMEM((PK,), jnp.int32),
            pltpu.VMEM((PK, D), jnp.float32),
            pltpu.VMEM((PK, D), jnp.float32),
            pltpu.VMEM((PK,), jnp.int32),
            pltpu.VMEM((PK,), jnp.int32),
            pltpu.VMEM((PK, 128), jnp.float32),
            pltpu.VMEM_SHARED((S, D), jnp.float32),
            pltpu.VMEM_SHARED((S, D), jnp.float32),
            pltpu.VMEM_SHARED((16, 128), jnp.float32),
            pltpu.SemaphoreType.DMA,
        ],
    )(_pool_body)


def _pool_body(h1_hbm, h2_hbm, seg_hbm, zrow_hbm, eye_hbm,
               out1_hbm, out2_hbm, outc_hbm,
               seg_v, r1_v, r2_v, rr_v, cc_v, oneh_v, acc1, acc2, accc, sem):
    c = lax.axis_index("c")
    s = lax.axis_index("s")
    t = c * NS + s
    pltpu.sync_copy(zrow_hbm, acc1.at[pl.ds(s * SPT, SPT)])
    pltpu.sync_copy(zrow_hbm, acc2.at[pl.ds(s * SPT, SPT)])

    @pl.when(s < 2)
    def _():
        pltpu.sync_copy(zrow_hbm.at[pl.ds(0, 8)], accc.at[pl.ds(s * 8, 8)])

    plsc.subcore_barrier()

    for k in range(4):
        cid = k * NW + t

        @pl.when(cid < PCH)
        def _():
            base = pl.multiple_of(cid * PK, 8)
            pltpu.sync_copy(seg_hbm.at[pl.ds(base, PK)], seg_v)
            pltpu.sync_copy(h1_hbm.at[pl.ds(base, PK)], r1_v)
            pltpu.sync_copy(h2_hbm.at[pl.ds(base, PK)], r2_v)
            pltpu.sync_copy(r1_v, acc1.at[seg_v], add=True)
            pltpu.sync_copy(r2_v, acc2.at[seg_v], add=True)
            for q in range(PK // 16):
                seg16 = seg_v[pl.ds(16 * q, 16)]
                rr_v[pl.ds(16 * q, 16)] = lax.shift_right_logical(seg16, 7)
                cc_v[pl.ds(16 * q, 16)] = lax.bitwise_and(seg16, 127)
            pltpu.async_copy(eye_hbm.at[cc_v], oneh_v, sem).wait()
            pltpu.sync_copy(oneh_v, accc.at[rr_v], add=True)

    plsc.subcore_barrier()
    pltpu.sync_copy(acc1.at[pl.ds(s * SPT, SPT)], out1_hbm.at[c, pl.ds(s * SPT, SPT)])
    pltpu.sync_copy(acc2.at[pl.ds(s * SPT, SPT)], out2_hbm.at[c, pl.ds(s * SPT, SPT)])

    @pl.when(s < 2)
    def _():
        pltpu.sync_copy(accc.at[pl.ds(s * 8, 8)], outc_hbm.at[c, pl.ds(s * 8, 8)])


def _mm1_body(x_ref, w_ref, d0_ref, d1_ref, hs_ref, dis_ref):
    deg = d0_ref[0] + d1_ref[0] + 1.0
    dis = lax.rsqrt(deg)
    m = jnp.dot(x_ref[...], w_ref[...], preferred_element_type=jnp.float32)
    hs_ref[...] = m * dis
    dis_ref[...] = jnp.broadcast_to(dis, (RB, 16))


def _ep1_body(p0_ref, p1_ref, hs_ref, dis_ref, b_ref, g_ref, stats_ref):
    g = dis_ref[:, 0:1] * (p0_ref[0] + p1_ref[0] + hs_ref[...]) + b_ref[...]
    g_ref[...] = g

    @pl.when(pl.program_id(0) == 0)
    def _():
        stats_ref[...] = jnp.zeros_like(stats_ref)

    upd = jnp.concatenate(
        [jnp.sum(g, axis=0, keepdims=True),
         jnp.sum(g * g, axis=0, keepdims=True),
         jnp.zeros((6, D), jnp.float32)], axis=0)
    stats_ref[...] += upd


def _bn_mm2_body(g_ref, stats_ref, gamma_ref, beta_ref, w_ref, dis_ref,
                 h1_ref, hs2_ref):
    inv_n = 1.0 / N
    mean = stats_ref[0:1, :] * inv_n
    var = stats_ref[1:2, :] * inv_n - mean * mean
    inv = lax.rsqrt(var + 1e-5)
    h1 = (g_ref[...] - mean) * inv * gamma_ref[...] + beta_ref[...]
    h1 = jnp.maximum(h1, 0.0)
    h1_ref[...] = h1
    m = jnp.dot(h1, w_ref[...], preferred_element_type=jnp.float32)
    hs2_ref[...] = m * dis_ref[:, 0:1]


def _ep2_body(p0_ref, p1_ref, hs_ref, dis_ref, b_ref, h2_ref):
    g = dis_ref[:, 0:1] * (p0_ref[0] + p1_ref[0] + hs_ref[...]) + b_ref[...]
    h2_ref[...] = jnp.maximum(g, 0.0)


def _final_body(s1_ref, s2_ref, c0_ref, c1_ref, out_ref):
    cnt = jnp.maximum(c0_ref[0] + c1_ref[0], 1.0)
    out_ref[:, :D] = (s1_ref[0] + s1_ref[1]) / cnt
    out_ref[:, D:] = (s2_ref[0] + s2_ref[1]) / cnt


def _row_spec(shape):
    return pl.BlockSpec(shape, lambda i: (i, 0))


def _fixed_spec(shape):
    return pl.BlockSpec(shape, lambda i: tuple(0 for _ in shape))


def _part_spec(core):
    return pl.BlockSpec((1, RB, D), lambda i, c=core: (c, i, 0))


def kernel(x, sub_edge_index, node_to_subgraph, W1, b1, gamma, beta, W2, b2):
    src = sub_edge_index[0]
    dst = sub_edge_index[1]
    zrows = jnp.zeros((RPT, D), jnp.float32)
    eye = jnp.eye(128, dtype=jnp.float32)

    degp = _make_deg_kernel()(dst, eye,
                              jnp.zeros((8, 128), jnp.float32)).reshape(NC, NPAD, 1)

    hs1, dis = pl.pallas_call(
        _mm1_body,
        grid=(GRID,),
        in_specs=[_row_spec((RB, D)), _fixed_spec((D, D)),
                  pl.BlockSpec((1, RB, 1), lambda i: (0, i, 0)),
                  pl.BlockSpec((1, RB, 1), lambda i: (1, i, 0))],
        out_specs=[_row_spec((RB, D)), _row_spec((RB, 16))],
        out_shape=[jax.ShapeDtypeStruct((N, D), jnp.float32),
                   jax.ShapeDtypeStruct((N, 16), jnp.float32)],
    )(x, W1, degp, degp)

    parts1 = _make_agg_kernel()(hs1, src, dst, zrows)

    g1, stats = pl.pallas_call(
        _ep1_body,
        grid=(GRID,),
        in_specs=[_part_spec(0), _part_spec(1), _row_spec((RB, D)),
                  _row_spec((RB, 16)), _fixed_spec((1, D))],
        out_specs=[_row_spec((RB, D)), _fixed_spec((8, D))],
        out_shape=[jax.ShapeDtypeStruct((N, D), jnp.float32),
                   jax.ShapeDtypeStruct((8, D), jnp.float32)],
    )(parts1, parts1, hs1, dis, b1.reshape(1, D))

    h1, hs2 = pl.pallas_call(
        _bn_mm2_body,
        grid=(GRID,),
        in_specs=[_row_spec((RB, D)), _fixed_spec((8, D)), _fixed_spec((1, D)),
                  _fixed_spec((1, D)), _fixed_spec((D, D)), _row_spec((RB, 16))],
        out_specs=[_row_spec((RB, D)), _row_spec((RB, D))],
        out_shape=[jax.ShapeDtypeStruct((N, D), jnp.float32),
                   jax.ShapeDtypeStruct((N, D), jnp.float32)],
    )(g1, stats, gamma.reshape(1, D), beta.reshape(1, D), W2, dis)

    parts2 = _make_agg_kernel()(hs2, src, dst, zrows)

    h2 = pl.pallas_call(
        _ep2_body,
        grid=(GRID,),
        in_specs=[_part_spec(0), _part_spec(1), _row_spec((RB, D)),
                  _row_spec((RB, 16)), _fixed_spec((1, D))],
        out_specs=_row_spec((RB, D)),
        out_shape=jax.ShapeDtypeStruct((N, D), jnp.float32),
    )(parts2, parts2, hs2, dis, b2.reshape(1, D))

    psum1, psum2, pcnt = _make_pool_kernel()(
        h1, h2, node_to_subgraph, jnp.zeros((SPT, D), jnp.float32), eye)
    pcnt = pcnt.reshape(NC, 16 * 128, 1)

    out = pl.pallas_call(
        _final_body,
        grid=(1,),
        in_specs=[pl.BlockSpec((NC, S, D), lambda i: (0, 0, 0)),
                  pl.BlockSpec((NC, S, D), lambda i: (0, 0, 0)),
                  pl.BlockSpec((1, S, 1), lambda i: (0, 0, 0)),
                  pl.BlockSpec((1, S, 1), lambda i: (1, 0, 0))],
        out_specs=pl.BlockSpec((S, 2 * D), lambda i: (0, 0)),
        out_shape=jax.ShapeDtypeStruct((S, 2 * D), jnp.float32),
    )(psum1, psum2, pcnt, pcnt)
    return out


# speedup vs baseline: 15.1469x; 1.2280x over previous
import functools

import jax
import jax.numpy as jnp
from jax import lax
from jax.experimental import pallas as pl
from jax.experimental.pallas import tpu as pltpu
from jax.experimental.pallas import tpu_sc as plsc

N = 10000
E = 320000
D = 128
S = 512

NC = 2
NS = 16
NW = NC * NS
ET = E // NW
EK = 80
NCHUNK = ET // EK
NPAD = 10240
RPT = NPAD // NS
RB = 400
GRID = N // RB
PK = 80
PCH = N // PK
SPT = S // NS


def _mesh():
    return plsc.VectorSubcoreMesh(
        core_axis_name="c", subcore_axis_name="s", num_cores=NC, num_subcores=NS)


NPR = NPAD // 128


@functools.cache
def _make_deg_kernel():
    return functools.partial(
        pl.kernel,
        out_type=jax.ShapeDtypeStruct((NC, NS * NPR, 128), jnp.float32),
        mesh=_mesh(),
        scratch_types=[
            pltpu.VMEM((EK,), jnp.int32),
            pltpu.VMEM((EK,), jnp.int32),
            pltpu.VMEM((EK,), jnp.int32),
            pltpu.VMEM((EK,), jnp.int32),
            pltpu.VMEM((EK,), jnp.int32),
            pltpu.VMEM((EK,), jnp.int32),
            pltpu.VMEM((EK, 128), jnp.float32),
            pltpu.VMEM((EK, 128), jnp.float32),
            pltpu.VMEM_SHARED((NS * NPR, 128), jnp.float32),
            pltpu.SemaphoreType.DMA,
            pltpu.SemaphoreType.DMA,
        ],
    )(_deg_body)


def _deg_body(dst_hbm, eye_hbm, zeros_hbm, out_hbm,
              idx0_v, r0_v, c0_v, idx1_v, r1_v, c1_v, oneh0_v, oneh1_v,
              acc, sem0, sem1):
    c = lax.axis_index("c")
    s = lax.axis_index("s")
    t = c * NS + s
    pltpu.sync_copy(zeros_hbm.at[pl.ds(0, NPR)], acc.at[pl.ds(s * NPR, NPR)])

    def load_and_gather(j, idx_v, r_v, c_v, oneh_v, sem):
        base = pl.multiple_of(t * ET + j * EK, 8)
        pltpu.sync_copy(dst_hbm.at[pl.ds(base, EK)], idx_v)
        for k in range(EK // 16):
            idx16 = idx_v[pl.ds(16 * k, 16)]
            r_v[pl.ds(16 * k, 16)] = (
                lax.shift_right_logical(idx16, 7) + s * NPR)
            c_v[pl.ds(16 * k, 16)] = lax.bitwise_and(idx16, 127) + t * 128
        pltpu.async_copy(eye_hbm.at[c_v], oneh_v, sem)

    load_and_gather(0, idx0_v, r0_v, c0_v, oneh0_v, sem0)

    def body(i, carry):
        j0 = 2 * i
        load_and_gather(j0 + 1, idx1_v, r1_v, c1_v, oneh1_v, sem1)
        pltpu.make_async_copy(eye_hbm.at[c0_v], oneh0_v, sem0).wait()
        pltpu.sync_copy(oneh0_v, acc.at[r0_v], add=True)
        load_and_gather(j0 + 2, idx0_v, r0_v, c0_v, oneh0_v, sem0)
        pltpu.make_async_copy(eye_hbm.at[c1_v], oneh1_v, sem1).wait()
        pltpu.sync_copy(oneh1_v, acc.at[r1_v], add=True)
        return carry

    lax.fori_loop(0, (NCHUNK - 1) // 2, body, 0)
    pltpu.make_async_copy(eye_hbm.at[c0_v], oneh0_v, sem0).wait()
    pltpu.sync_copy(oneh0_v, acc.at[r0_v], add=True)
    pltpu.sync_copy(acc.at[pl.ds(s * NPR, NPR)],
                    out_hbm.at[c, pl.ds(s * NPR, NPR)])


@functools.cache
def _make_agg_kernel():
    return functools.partial(
        pl.kernel,
        out_type=jax.ShapeDtypeStruct((NC, NPAD, D), jnp.float32),
        mesh=_mesh(),
        scratch_types=[
            pltpu.VMEM((EK,), jnp.int32),
            pltpu.VMEM((EK,), jnp.int32),
            pltpu.VMEM((EK,), jnp.int32),
            pltpu.VMEM((EK,), jnp.int32),
            pltpu.VMEM((EK, D), jnp.float32),
            pltpu.VMEM((EK, D), jnp.float32),
            pltpu.VMEM_SHARED((NPAD, D), jnp.float32),
            pltpu.SemaphoreType.DMA,
            pltpu.SemaphoreType.DMA,
        ],
    )(_agg_body)


def _agg_body(hs_hbm, src_hbm, dst_hbm, zeros_hbm, out_hbm,
              src0_v, dst0_v, src1_v, dst1_v, rows0_v, rows1_v,
              acc, sem0, sem1):
    c = lax.axis_index("c")
    s = lax.axis_index("s")
    t = c * NS + s
    pltpu.sync_copy(zeros_hbm, acc.at[pl.ds(s * RPT, RPT)])
    plsc.subcore_barrier()

    def load_and_gather(j, idx_s, idx_d, rows, sem):
        base = pl.multiple_of(t * ET + j * EK, 8)
        pltpu.sync_copy(src_hbm.at[pl.ds(base, EK)], idx_s)
        pltpu.sync_copy(dst_hbm.at[pl.ds(base, EK)], idx_d)
        pltpu.async_copy(hs_hbm.at[idx_s], rows, sem)

    load_and_gather(0, src0_v, dst0_v, rows0_v, sem0)

    def body(i, carry):
        j0 = 2 * i
        load_and_gather(j0 + 1, src1_v, dst1_v, rows1_v, sem1)
        pltpu.make_async_copy(hs_hbm.at[src0_v], rows0_v, sem0).wait()
        pltpu.sync_copy(rows0_v, acc.at[dst0_v], add=True)
        load_and_gather(j0 + 2, src0_v, dst0_v, rows0_v, sem0)
        pltpu.make_async_copy(hs_hbm.at[src1_v], rows1_v, sem1).wait()
        pltpu.sync_copy(rows1_v, acc.at[dst1_v], add=True)
        return carry

    lax.fori_loop(0, (NCHUNK - 1) // 2, body, 0)
    pltpu.make_async_copy(hs_hbm.at[src0_v], rows0_v, sem0).wait()
    pltpu.sync_copy(rows0_v, acc.at[dst0_v], add=True)
    plsc.subcore_barrier()
    pltpu.sync_copy(acc.at[pl.ds(s * RPT, RPT)],
                    out_hbm.at[c, pl.ds(s * RPT, RPT)])


@functools.cache
def _make_pool_kernel():
    return functools.partial(
        pl.kernel,
        out_type=[
            jax.ShapeDtypeStruct((NC, S, D), jnp.float32),
            jax.ShapeDtypeStruct((NC, S, D), jnp.float32),
            jax.ShapeDtypeStruct((NC, 16, 128), jnp.float32),
        ],
        mesh=_mesh(),
        scratch_types=[
            pltpu.VMEM((PK,), jnp.int32),
            pltpu.VMEM((PK, D), jnp.float32),
            pltpu.VMEM((PK, D), jnp.float32),
            pltpu.VMEM((PK,), jnp.int32),
            pltpu.VMEM((PK,), jnp.int32),
            pltpu.VMEM((PK, 128), jnp.float32),
            pltpu.VMEM_SHARED((S, D), jnp.float32),
            pltpu.VMEM_SHARED((S, D), jnp.float32),
            pltpu.VMEM_SHARED((16, 128), jnp.float32),
            pltpu.SemaphoreType.DMA,
        ],
    )(_pool_body)


def _pool_body(h1_hbm, h2_hbm, seg_hbm, zrow_hbm, eye_hbm,
               out1_hbm, out2_hbm, outc_hbm,
               seg_v, r1_v, r2_v, rr_v, cc_v, oneh_v, acc1, acc2, accc, sem):
    c = lax.axis_index("c")
    s = lax.axis_index("s")
    t = c * NS + s
    pltpu.sync_copy(zrow_hbm, acc1.at[pl.ds(s * SPT, SPT)])
    pltpu.sync_copy(zrow_hbm, acc2.at[pl.ds(s * SPT, SPT)])

    @pl.when(s < 2)
    def _():
        pltpu.sync_copy(zrow_hbm.at[pl.ds(0, 8)], accc.at[pl.ds(s * 8, 8)])

    plsc.subcore_barrier()

    for k in range(4):
        cid = k * NW + t

        @pl.when(cid < PCH)
        def _():
            base = pl.multiple_of(cid * PK, 8)
            pltpu.sync_copy(seg_hbm.at[pl.ds(base, PK)], seg_v)
            pltpu.sync_copy(h1_hbm.at[pl.ds(base, PK)], r1_v)
            pltpu.sync_copy(h2_hbm.at[pl.ds(base, PK)], r2_v)
            pltpu.sync_copy(r1_v, acc1.at[seg_v], add=True)
            pltpu.sync_copy(r2_v, acc2.at[seg_v], add=True)
            for q in range(PK // 16):
                seg16 = seg_v[pl.ds(16 * q, 16)]
                rr_v[pl.ds(16 * q, 16)] = lax.shift_right_logical(seg16, 7)
                cc_v[pl.ds(16 * q, 16)] = (
                    lax.bitwise_and(seg16, 127) + t * 128)
            pltpu.async_copy(eye_hbm.at[cc_v], oneh_v, sem).wait()
            pltpu.sync_copy(oneh_v, accc.at[rr_v], add=True)

    plsc.subcore_barrier()
    pltpu.sync_copy(acc1.at[pl.ds(s * SPT, SPT)], out1_hbm.at[c, pl.ds(s * SPT, SPT)])
    pltpu.sync_copy(acc2.at[pl.ds(s * SPT, SPT)], out2_hbm.at[c, pl.ds(s * SPT, SPT)])

    @pl.when(s < 2)
    def _():
        pltpu.sync_copy(accc.at[pl.ds(s * 8, 8)], outc_hbm.at[c, pl.ds(s * 8, 8)])


def _degmerge_body(p_ref, out_ref):
    def body(i, acc):
        return acc + p_ref[pl.ds(i * NPR, NPR), :]

    out_ref[...] = lax.fori_loop(
        0, NW, body, jnp.zeros((NPR, 128), jnp.float32))


def _mm1_body(x_ref, w_ref, d_ref, hs_ref, dis_ref):
    deg = d_ref[...] + 1.0
    dis = lax.rsqrt(deg)
    m = jnp.dot(x_ref[...], w_ref[...], preferred_element_type=jnp.float32)
    hs_ref[...] = m * dis
    dis_ref[...] = jnp.broadcast_to(dis, (RB, 16))


def _ep1_body(p0_ref, p1_ref, hs_ref, dis_ref, b_ref, g_ref, stats_ref):
    g = dis_ref[:, 0:1] * (p0_ref[0] + p1_ref[0] + hs_ref[...]) + b_ref[...]
    g_ref[...] = g

    @pl.when(pl.program_id(0) == 0)
    def _():
        stats_ref[...] = jnp.zeros_like(stats_ref)

    upd = jnp.concatenate(
        [jnp.sum(g, axis=0, keepdims=True),
         jnp.sum(g * g, axis=0, keepdims=True),
         jnp.zeros((6, D), jnp.float32)], axis=0)
    stats_ref[...] += upd


def _bn_mm2_body(g_ref, stats_ref, gamma_ref, beta_ref, w_ref, dis_ref,
                 h1_ref, hs2_ref):
    inv_n = 1.0 / N
    mean = stats_ref[0:1, :] * inv_n
    var = stats_ref[1:2, :] * inv_n - mean * mean
    inv = lax.rsqrt(var + 1e-5)
    h1 = (g_ref[...] - mean) * inv * gamma_ref[...] + beta_ref[...]
    h1 = jnp.maximum(h1, 0.0)
    h1_ref[...] = h1
    m = jnp.dot(h1, w_ref[...], preferred_element_type=jnp.float32)
    hs2_ref[...] = m * dis_ref[:, 0:1]


def _ep2_body(p0_ref, p1_ref, hs_ref, dis_ref, b_ref, h2_ref):
    g = dis_ref[:, 0:1] * (p0_ref[0] + p1_ref[0] + hs_ref[...]) + b_ref[...]
    h2_ref[...] = jnp.maximum(g, 0.0)


def _final_body(s1_ref, s2_ref, c0_ref, c1_ref, out_ref):
    cnt = jnp.maximum(c0_ref[0] + c1_ref[0], 1.0)
    out_ref[:, :D] = (s1_ref[0] + s1_ref[1]) / cnt
    out_ref[:, D:] = (s2_ref[0] + s2_ref[1]) / cnt


def _row_spec(shape):
    return pl.BlockSpec(shape, lambda i: (i, 0))


def _fixed_spec(shape):
    return pl.BlockSpec(shape, lambda i: tuple(0 for _ in shape))


def _part_spec(core):
    return pl.BlockSpec((1, RB, D), lambda i, c=core: (c, i, 0))


def kernel(x, sub_edge_index, node_to_subgraph, W1, b1, gamma, beta, W2, b2):
    src = sub_edge_index[0]
    dst = sub_edge_index[1]
    zrows = jnp.zeros((RPT, D), jnp.float32)
    eye = jnp.tile(jnp.eye(128, dtype=jnp.float32), (NW, 1))

    degp = _make_deg_kernel()(dst, eye, zrows)
    degsum = pl.pallas_call(
        _degmerge_body,
        out_shape=jax.ShapeDtypeStruct((NPR, 128), jnp.float32),
    )(degp.reshape(NC * NS * NPR, 128))
    deg1 = degsum.reshape(NPAD, 1)

    hs1, dis = pl.pallas_call(
        _mm1_body,
        grid=(GRID,),
        in_specs=[_row_spec((RB, D)), _fixed_spec((D, D)),
                  pl.BlockSpec((RB, 1), lambda i: (i, 0))],
        out_specs=[_row_spec((RB, D)), _row_spec((RB, 16))],
        out_shape=[jax.ShapeDtypeStruct((N, D), jnp.float32),
                   jax.ShapeDtypeStruct((N, 16), jnp.float32)],
    )(x, W1, deg1)

    parts1 = _make_agg_kernel()(hs1, src, dst, zrows)

    g1, stats = pl.pallas_call(
        _ep1_body,
        grid=(GRID,),
        in_specs=[_part_spec(0), _part_spec(1), _row_spec((RB, D)),
                  _row_spec((RB, 16)), _fixed_spec((1, D))],
        out_specs=[_row_spec((RB, D)), _fixed_spec((8, D))],
        out_shape=[jax.ShapeDtypeStruct((N, D), jnp.float32),
                   jax.ShapeDtypeStruct((8, D), jnp.float32)],
    )(parts1, parts1, hs1, dis, b1.reshape(1, D))

    h1, hs2 = pl.pallas_call(
        _bn_mm2_body,
        grid=(GRID,),
        in_specs=[_row_spec((RB, D)), _fixed_spec((8, D)), _fixed_spec((1, D)),
                  _fixed_spec((1, D)), _fixed_spec((D, D)), _row_spec((RB, 16))],
        out_specs=[_row_spec((RB, D)), _row_spec((RB, D))],
        out_shape=[jax.ShapeDtypeStruct((N, D), jnp.float32),
                   jax.ShapeDtypeStruct((N, D), jnp.float32)],
    )(g1, stats, gamma.reshape(1, D), beta.reshape(1, D), W2, dis)

    parts2 = _make_agg_kernel()(hs2, src, dst, zrows)

    h2 = pl.pallas_call(
        _ep2_body,
        grid=(GRID,),
        in_specs=[_part_spec(0), _part_spec(1), _row_spec((RB, D)),
                  _row_spec((RB, 16)), _fixed_spec((1, D))],
        out_specs=_row_spec((RB, D)),
        out_shape=jax.ShapeDtypeStruct((N, D), jnp.float32),
    )(parts2, parts2, hs2, dis, b2.reshape(1, D))

    psum1, psum2, pcnt = _make_pool_kernel()(
        h1, h2, node_to_subgraph, jnp.zeros((SPT, D), jnp.float32), eye)
    pcnt = pcnt.reshape(NC, 16 * 128, 1)

    out = pl.pallas_call(
        _final_body,
        grid=(1,),
        in_specs=[pl.BlockSpec((NC, S, D), lambda i: (0, 0, 0)),
                  pl.BlockSpec((NC, S, D), lambda i: (0, 0, 0)),
                  pl.BlockSpec((1, S, 1), lambda i: (0, 0, 0)),
                  pl.BlockSpec((1, S, 1), lambda i: (1, 0, 0))],
        out_specs=pl.BlockSpec((S, 2 * D), lambda i: (0, 0)),
        out_shape=jax.ShapeDtypeStruct((S, 2 * D), jnp.float32),
    )(psum1, psum2, pcnt, pcnt)
    return out


# speedup vs baseline: 15.2073x; 1.0040x over previous
import functools

import jax
import jax.numpy as jnp
from jax import lax
from jax.experimental import pallas as pl
from jax.experimental.pallas import tpu as pltpu
from jax.experimental.pallas import tpu_sc as plsc

N = 10000
E = 320000
D = 128
S = 512

NC = 2
NS = 16
NW = NC * NS
ET = E // NW
EK = 80
NCHUNK = ET // EK
NPAD = 10240
RPT = NPAD // NS
RB = 400
GRID = N // RB
PK = 80
PCH = N // PK
SPT = S // NS


def _mesh():
    return plsc.VectorSubcoreMesh(
        core_axis_name="c", subcore_axis_name="s", num_cores=NC, num_subcores=NS)


NPR = NPAD // 128


@functools.cache
def _make_deg_kernel():
    return functools.partial(
        pl.kernel,
        out_type=jax.ShapeDtypeStruct((NC, NS * NPR, 128), jnp.float32),
        mesh=_mesh(),
        scratch_types=[
            pltpu.VMEM((EK,), jnp.int32),
            pltpu.VMEM((EK,), jnp.int32),
            pltpu.VMEM((EK,), jnp.int32),
            pltpu.VMEM((EK,), jnp.int32),
            pltpu.VMEM((EK,), jnp.int32),
            pltpu.VMEM((EK,), jnp.int32),
            pltpu.VMEM((EK, 128), jnp.float32),
            pltpu.VMEM((EK, 128), jnp.float32),
            pltpu.VMEM_SHARED((NS * NPR, 128), jnp.float32),
            pltpu.SemaphoreType.DMA,
            pltpu.SemaphoreType.DMA,
        ],
    )(_deg_body)


def _deg_body(dst_hbm, eye_hbm, zeros_hbm, out_hbm,
              idx0_v, r0_v, c0_v, idx1_v, r1_v, c1_v, oneh0_v, oneh1_v,
              acc, sem0, sem1):
    c = lax.axis_index("c")
    s = lax.axis_index("s")
    t = c * NS + s
    pltpu.sync_copy(zeros_hbm.at[pl.ds(0, NPR)], acc.at[pl.ds(s * NPR, NPR)])

    def load_and_gather(j, idx_v, r_v, c_v, oneh_v, sem):
        base = pl.multiple_of(t * ET + j * EK, 8)
        pltpu.sync_copy(dst_hbm.at[pl.ds(base, EK)], idx_v)
        for k in range(EK // 16):
            idx16 = idx_v[pl.ds(16 * k, 16)]
            r_v[pl.ds(16 * k, 16)] = (
                lax.shift_right_logical(idx16, 7) + s * NPR)
            c_v[pl.ds(16 * k, 16)] = lax.bitwise_and(idx16, 127) + t * 128
        pltpu.async_copy(eye_hbm.at[c_v], oneh_v, sem)

    load_and_gather(0, idx0_v, r0_v, c0_v, oneh0_v, sem0)

    def body(i, carry):
        j0 = 2 * i
        load_and_gather(j0 + 1, idx1_v, r1_v, c1_v, oneh1_v, sem1)
        pltpu.make_async_copy(eye_hbm.at[c0_v], oneh0_v, sem0).wait()
        pltpu.sync_copy(oneh0_v, acc.at[r0_v], add=True)
        load_and_gather(j0 + 2, idx0_v, r0_v, c0_v, oneh0_v, sem0)
        pltpu.make_async_copy(eye_hbm.at[c1_v], oneh1_v, sem1).wait()
        pltpu.sync_copy(oneh1_v, acc.at[r1_v], add=True)
        return carry

    lax.fori_loop(0, (NCHUNK - 1) // 2, body, 0)
    pltpu.make_async_copy(eye_hbm.at[c0_v], oneh0_v, sem0).wait()
    pltpu.sync_copy(oneh0_v, acc.at[r0_v], add=True)
    pltpu.sync_copy(acc.at[pl.ds(s * NPR, NPR)],
                    out_hbm.at[c, pl.ds(s * NPR, NPR)])


@functools.cache
def _make_agg_kernel():
    return functools.partial(
        pl.kernel,
        out_type=jax.ShapeDtypeStruct((NC, NPAD, D), jnp.float32),
        mesh=_mesh(),
        scratch_types=(
            [pltpu.VMEM((EK,), jnp.int32)] * 8
            + [pltpu.VMEM((EK, D), jnp.float32)] * 4
            + [pltpu.VMEM_SHARED((NPAD, D), jnp.float32)]
            + [pltpu.SemaphoreType.DMA] * 4
        ),
    )(_agg_body)


_NBUF = 4


def _agg_body(hs_hbm, src_hbm, dst_hbm, zeros_hbm, out_hbm, *refs):
    src_vs = refs[0:4]
    dst_vs = refs[4:8]
    rows_vs = refs[8:12]
    acc = refs[12]
    sems = refs[13:17]
    c = lax.axis_index("c")
    s = lax.axis_index("s")
    t = c * NS + s
    pltpu.sync_copy(zeros_hbm, acc.at[pl.ds(s * RPT, RPT)])
    plsc.subcore_barrier()

    def load_and_gather(j, b):
        base = pl.multiple_of(t * ET + j * EK, 8)
        pltpu.sync_copy(src_hbm.at[pl.ds(base, EK)], src_vs[b])
        pltpu.sync_copy(dst_hbm.at[pl.ds(base, EK)], dst_vs[b])
        pltpu.async_copy(hs_hbm.at[src_vs[b]], rows_vs[b], sems[b])

    def drain_and_scatter(b):
        pltpu.make_async_copy(hs_hbm.at[src_vs[b]], rows_vs[b], sems[b]).wait()
        pltpu.sync_copy(rows_vs[b], acc.at[dst_vs[b]], add=True)

    PRO = _NBUF - 1
    for j in range(PRO):
        load_and_gather(j, j)

    def body(i, carry):
        j0 = _NBUF * i
        for b in range(_NBUF):
            drain_and_scatter(b)
            load_and_gather(j0 + b + PRO, (b + PRO) % _NBUF)
        return carry

    n_full = (NCHUNK - PRO - _NBUF) // _NBUF + 1
    lax.fori_loop(0, n_full, body, 0)
    for j in range(_NBUF * n_full, NCHUNK):
        drain_and_scatter(j % _NBUF)
        if j + PRO < NCHUNK:
            load_and_gather(j + PRO, (j + PRO) % _NBUF)
    plsc.subcore_barrier()
    pltpu.sync_copy(acc.at[pl.ds(s * RPT, RPT)],
                    out_hbm.at[c, pl.ds(s * RPT, RPT)])


@functools.cache
def _make_pool_kernel():
    return functools.partial(
        pl.kernel,
        out_type=[
            jax.ShapeDtypeStruct((NC, S, D), jnp.float32),
            jax.ShapeDtypeStruct((NC, S, D), jnp.float32),
            jax.ShapeDtypeStruct((NC, 16, 128), jnp.float32),
        ],
        mesh=_mesh(),
        scratch_types=[
            pltpu.VMEM((PK,), jnp.int32),
            pltpu.VMEM((PK, D), jnp.float32),
            pltpu.VMEM((PK, D), jnp.float32),
            pltpu.VMEM((PK,), jnp.int32),
            pltpu.VMEM((PK,), jnp.int32),
            pltpu.VMEM((PK, 128), jnp.float32),
            pltpu.VMEM_SHARED((S, D), jnp.float32),
            pltpu.VMEM_SHARED((S, D), jnp.float32),
            pltpu.VMEM_SHARED((16, 128), jnp.float32),
            pltpu.SemaphoreType.DMA,
        ],
    )(_pool_body)


def _pool_body(h1_hbm, h2_hbm, seg_hbm, zrow_hbm, eye_hbm,
               out1_hbm, out2_hbm, outc_hbm,
               seg_v, r1_v, r2_v, rr_v, cc_v, oneh_v, acc1, acc2, accc, sem):
    c = lax.axis_index("c")
    s = lax.axis_index("s")
    t = c * NS + s
    pltpu.sync_copy(zrow_hbm, acc1.at[pl.ds(s * SPT, SPT)])
    pltpu.sync_copy(zrow_hbm, acc2.at[pl.ds(s * SPT, SPT)])

    @pl.when(s < 2)
    def _():
        pltpu.sync_copy(zrow_hbm.at[pl.ds(0, 8)], accc.at[pl.ds(s * 8, 8)])

    plsc.subcore_barrier()

    for k in range(4):
        cid = k * NW + t

        @pl.when(cid < PCH)
        def _():
            base = pl.multiple_of(cid * PK, 8)
            pltpu.sync_copy(seg_hbm.at[pl.ds(base, PK)], seg_v)
            pltpu.sync_copy(h1_hbm.at[pl.ds(base, PK)], r1_v)
            pltpu.sync_copy(h2_hbm.at[pl.ds(base, PK)], r2_v)
            pltpu.sync_copy(r1_v, acc1.at[seg_v], add=True)
            pltpu.sync_copy(r2_v, acc2.at[seg_v], add=True)
            for q in range(PK // 16):
                seg16 = seg_v[pl.ds(16 * q, 16)]
                rr_v[pl.ds(16 * q, 16)] = lax.shift_right_logical(seg16, 7)
                cc_v[pl.ds(16 * q, 16)] = (
                    lax.bitwise_and(seg16, 127) + t * 128)
            pltpu.async_copy(eye_hbm.at[cc_v], oneh_v, sem).wait()
            pltpu.sync_copy(oneh_v, accc.at[rr_v], add=True)

    plsc.subcore_barrier()
    pltpu.sync_copy(acc1.at[pl.ds(s * SPT, SPT)], out1_hbm.at[c, pl.ds(s * SPT, SPT)])
    pltpu.sync_copy(acc2.at[pl.ds(s * SPT, SPT)], out2_hbm.at[c, pl.ds(s * SPT, SPT)])

    @pl.when(s < 2)
    def _():
        pltpu.sync_copy(accc.at[pl.ds(s * 8, 8)], outc_hbm.at[c, pl.ds(s * 8, 8)])


def _degmerge_body(p_ref, out_ref):
    def body(i, acc):
        return acc + p_ref[pl.ds(i * NPR, NPR), :]

    out_ref[...] = lax.fori_loop(
        0, NW, body, jnp.zeros((NPR, 128), jnp.float32))


def _mm1_body(x_ref, w_ref, d_ref, hs_ref, dis_ref):
    deg = d_ref[...] + 1.0
    dis = lax.rsqrt(deg)
    m = jnp.dot(x_ref[...], w_ref[...], preferred_element_type=jnp.float32)
    hs_ref[...] = m * dis
    dis_ref[...] = jnp.broadcast_to(dis, (RB, 16))


def _ep1_body(p0_ref, p1_ref, hs_ref, dis_ref, b_ref, g_ref, stats_ref):
    g = dis_ref[:, 0:1] * (p0_ref[0] + p1_ref[0] + hs_ref[...]) + b_ref[...]
    g_ref[...] = g

    @pl.when(pl.program_id(0) == 0)
    def _():
        stats_ref[...] = jnp.zeros_like(stats_ref)

    upd = jnp.concatenate(
        [jnp.sum(g, axis=0, keepdims=True),
         jnp.sum(g * g, axis=0, keepdims=True),
         jnp.zeros((6, D), jnp.float32)], axis=0)
    stats_ref[...] += upd


def _bn_mm2_body(g_ref, stats_ref, gamma_ref, beta_ref, w_ref, dis_ref,
                 h1_ref, hs2_ref):
    inv_n = 1.0 / N
    mean = stats_ref[0:1, :] * inv_n
    var = stats_ref[1:2, :] * inv_n - mean * mean
    inv = lax.rsqrt(var + 1e-5)
    h1 = (g_ref[...] - mean) * inv * gamma_ref[...] + beta_ref[...]
    h1 = jnp.maximum(h1, 0.0)
    h1_ref[...] = h1
    m = jnp.dot(h1, w_ref[...], preferred_element_type=jnp.float32)
    hs2_ref[...] = m * dis_ref[:, 0:1]


def _ep2_body(p0_ref, p1_ref, hs_ref, dis_ref, b_ref, h2_ref):
    g = dis_ref[:, 0:1] * (p0_ref[0] + p1_ref[0] + hs_ref[...]) + b_ref[...]
    h2_ref[...] = jnp.maximum(g, 0.0)


def _final_body(s1_ref, s2_ref, c0_ref, c1_ref, out_ref):
    cnt = jnp.maximum(c0_ref[0] + c1_ref[0], 1.0)
    out_ref[:, :D] = (s1_ref[0] + s1_ref[1]) / cnt
    out_ref[:, D:] = (s2_ref[0] + s2_ref[1]) / cnt


def _row_spec(shape):
    return pl.BlockSpec(shape, lambda i: (i, 0))


def _fixed_spec(shape):
    return pl.BlockSpec(shape, lambda i: tuple(0 for _ in shape))


def _part_spec(core):
    return pl.BlockSpec((1, RB, D), lambda i, c=core: (c, i, 0))


def kernel(x, sub_edge_index, node_to_subgraph, W1, b1, gamma, beta, W2, b2):
    src = sub_edge_index[0]
    dst = sub_edge_index[1]
    zrows = jnp.zeros((RPT, D), jnp.float32)
    eye = jnp.tile(jnp.eye(128, dtype=jnp.float32), (NW, 1))

    degp = _make_deg_kernel()(dst, eye, zrows)
    degsum = pl.pallas_call(
        _degmerge_body,
        out_shape=jax.ShapeDtypeStruct((NPR, 128), jnp.float32),
    )(degp.reshape(NC * NS * NPR, 128))
    deg1 = degsum.reshape(NPAD, 1)

    hs1, dis = pl.pallas_call(
        _mm1_body,
        grid=(GRID,),
        in_specs=[_row_spec((RB, D)), _fixed_spec((D, D)),
                  pl.BlockSpec((RB, 1), lambda i: (i, 0))],
        out_specs=[_row_spec((RB, D)), _row_spec((RB, 16))],
        out_shape=[jax.ShapeDtypeStruct((N, D), jnp.float32),
                   jax.ShapeDtypeStruct((N, 16), jnp.float32)],
    )(x, W1, deg1)

    parts1 = _make_agg_kernel()(hs1, src, dst, zrows)

    g1, stats = pl.pallas_call(
        _ep1_body,
        grid=(GRID,),
        in_specs=[_part_spec(0), _part_spec(1), _row_spec((RB, D)),
                  _row_spec((RB, 16)), _fixed_spec((1, D))],
        out_specs=[_row_spec((RB, D)), _fixed_spec((8, D))],
        out_shape=[jax.ShapeDtypeStruct((N, D), jnp.float32),
                   jax.ShapeDtypeStruct((8, D), jnp.float32)],
    )(parts1, parts1, hs1, dis, b1.reshape(1, D))

    h1, hs2 = pl.pallas_call(
        _bn_mm2_body,
        grid=(GRID,),
        in_specs=[_row_spec((RB, D)), _fixed_spec((8, D)), _fixed_spec((1, D)),
                  _fixed_spec((1, D)), _fixed_spec((D, D)), _row_spec((RB, 16))],
        out_specs=[_row_spec((RB, D)), _row_spec((RB, D))],
        out_shape=[jax.ShapeDtypeStruct((N, D), jnp.float32),
                   jax.ShapeDtypeStruct((N, D), jnp.float32)],
    )(g1, stats, gamma.reshape(1, D), beta.reshape(1, D), W2, dis)

    parts2 = _make_agg_kernel()(hs2, src, dst, zrows)

    h2 = pl.pallas_call(
        _ep2_body,
        grid=(GRID,),
        in_specs=[_part_spec(0), _part_spec(1), _row_spec((RB, D)),
                  _row_spec((RB, 16)), _fixed_spec((1, D))],
        out_specs=_row_spec((RB, D)),
        out_shape=jax.ShapeDtypeStruct((N, D), jnp.float32),
    )(parts2, parts2, hs2, dis, b2.reshape(1, D))

    psum1, psum2, pcnt = _make_pool_kernel()(
        h1, h2, node_to_subgraph, jnp.zeros((SPT, D), jnp.float32), eye)
    pcnt = pcnt.reshape(NC, 16 * 128, 1)

    out = pl.pallas_call(
        _final_body,
        grid=(1,),
        in_specs=[pl.BlockSpec((NC, S, D), lambda i: (0, 0, 0)),
                  pl.BlockSpec((NC, S, D), lambda i: (0, 0, 0)),
                  pl.BlockSpec((1, S, 1), lambda i: (0, 0, 0)),
                  pl.BlockSpec((1, S, 1), lambda i: (1, 0, 0))],
        out_specs=pl.BlockSpec((S, 2 * D), lambda i: (0, 0)),
        out_shape=jax.ShapeDtypeStruct((S, 2 * D), jnp.float32),
    )(psum1, psum2, pcnt, pcnt)
    return out


# speedup vs baseline: 16.9654x; 1.1156x over previous
import functools

import jax
import jax.numpy as jnp
from jax import lax
from jax.experimental import pallas as pl
from jax.experimental.pallas import tpu as pltpu
from jax.experimental.pallas import tpu_sc as plsc

N = 10000
E = 320000
D = 128
S = 512

NC = 2
NS = 16
NW = NC * NS
ET = E // NW
EK = 80
NCHUNK = ET // EK
NPAD = 10240
RPT = NPAD // NS
RB = 400
GRID = N // RB
PK = 80
PCH = N // PK
SPT = S // NS


def _mesh():
    return plsc.VectorSubcoreMesh(
        core_axis_name="c", subcore_axis_name="s", num_cores=NC, num_subcores=NS)


NPR = NPAD // 128


@functools.cache
def _make_deg_kernel():
    return functools.partial(
        pl.kernel,
        out_type=jax.ShapeDtypeStruct((NC, NS * NPR, 128), jnp.float32),
        mesh=_mesh(),
        scratch_types=[
            pltpu.VMEM((EK,), jnp.int32),
            pltpu.VMEM((EK,), jnp.int32),
            pltpu.VMEM((EK,), jnp.int32),
            pltpu.VMEM((EK,), jnp.int32),
            pltpu.VMEM((EK,), jnp.int32),
            pltpu.VMEM((EK,), jnp.int32),
            pltpu.VMEM((EK, 128), jnp.float32),
            pltpu.VMEM((EK, 128), jnp.float32),
            pltpu.VMEM_SHARED((NS * NPR, 128), jnp.float32),
            pltpu.SemaphoreType.DMA,
            pltpu.SemaphoreType.DMA,
        ],
    )(_deg_body)


def _deg_body(dst_hbm, eye_hbm, zeros_hbm, out_hbm,
              idx0_v, r0_v, c0_v, idx1_v, r1_v, c1_v, oneh0_v, oneh1_v,
              acc, sem0, sem1):
    c = lax.axis_index("c")
    s = lax.axis_index("s")
    t = c * NS + s
    pltpu.sync_copy(zeros_hbm.at[pl.ds(0, NPR)], acc.at[pl.ds(s * NPR, NPR)])

    def load_and_gather(j, idx_v, r_v, c_v, oneh_v, sem):
        base = pl.multiple_of(t * ET + j * EK, 8)
        pltpu.sync_copy(dst_hbm.at[pl.ds(base, EK)], idx_v)
        for k in range(EK // 16):
            idx16 = idx_v[pl.ds(16 * k, 16)]
            r_v[pl.ds(16 * k, 16)] = (
                lax.shift_right_logical(idx16, 7) + s * NPR)
            c_v[pl.ds(16 * k, 16)] = lax.bitwise_and(idx16, 127) + t * 128
        pltpu.async_copy(eye_hbm.at[c_v], oneh_v, sem)

    load_and_gather(0, idx0_v, r0_v, c0_v, oneh0_v, sem0)

    def body(i, carry):
        j0 = 2 * i
        load_and_gather(j0 + 1, idx1_v, r1_v, c1_v, oneh1_v, sem1)
        pltpu.make_async_copy(eye_hbm.at[c0_v], oneh0_v, sem0).wait()
        pltpu.sync_copy(oneh0_v, acc.at[r0_v], add=True)
        load_and_gather(j0 + 2, idx0_v, r0_v, c0_v, oneh0_v, sem0)
        pltpu.make_async_copy(eye_hbm.at[c1_v], oneh1_v, sem1).wait()
        pltpu.sync_copy(oneh1_v, acc.at[r1_v], add=True)
        return carry

    lax.fori_loop(0, (NCHUNK - 1) // 2, body, 0)
    pltpu.make_async_copy(eye_hbm.at[c0_v], oneh0_v, sem0).wait()
    pltpu.sync_copy(oneh0_v, acc.at[r0_v], add=True)
    pltpu.sync_copy(acc.at[pl.ds(s * NPR, NPR)],
                    out_hbm.at[c, pl.ds(s * NPR, NPR)])


_NBUF = 4
_PRO = 2


@functools.cache
def _make_agg_kernel():
    return functools.partial(
        pl.kernel,
        out_type=jax.ShapeDtypeStruct((NC, NPAD, D), jnp.float32),
        mesh=_mesh(),
        scratch_types=(
            [pltpu.VMEM((EK,), jnp.int32)] * (2 * _NBUF)
            + [pltpu.VMEM((EK, D), jnp.float32)] * _NBUF
            + [pltpu.VMEM_SHARED((NPAD, D), jnp.float32)]
            + [pltpu.SemaphoreType.DMA] * (2 * _NBUF)
        ),
    )(_agg_body)


def _agg_body(hs_hbm, src_hbm, dst_hbm, zeros_hbm, out_hbm, *refs):
    src_vs = refs[0:_NBUF]
    dst_vs = refs[_NBUF:2 * _NBUF]
    rows_vs = refs[2 * _NBUF:3 * _NBUF]
    acc = refs[3 * _NBUF]
    sems = refs[3 * _NBUF + 1:4 * _NBUF + 1]
    sc_sems = refs[4 * _NBUF + 1:5 * _NBUF + 1]
    c = lax.axis_index("c")
    s = lax.axis_index("s")
    t = c * NS + s
    pltpu.sync_copy(zeros_hbm, acc.at[pl.ds(s * RPT, RPT)])
    plsc.subcore_barrier()

    def load_and_gather(j, b):
        base = pl.multiple_of(t * ET + j * EK, 8)
        pltpu.sync_copy(src_hbm.at[pl.ds(base, EK)], src_vs[b])
        pltpu.sync_copy(dst_hbm.at[pl.ds(base, EK)], dst_vs[b])
        pltpu.async_copy(hs_hbm.at[src_vs[b]], rows_vs[b], sems[b])

    def drain_and_scatter(b):
        pltpu.make_async_copy(hs_hbm.at[src_vs[b]], rows_vs[b], sems[b]).wait()
        pltpu.async_copy(rows_vs[b], acc.at[dst_vs[b]], sc_sems[b], add=True)

    def wait_scatter(b):
        pltpu.make_async_copy(rows_vs[b], acc.at[dst_vs[b]], sc_sems[b]).wait()

    PRO = _PRO
    for j in range(PRO):
        load_and_gather(j, j)

    def body(i, carry):
        j0 = _NBUF * i
        for b in range(_NBUF):
            drain_and_scatter(b)
            bn = (b + PRO) % _NBUF
            jn = j0 + b + PRO

            @pl.when(jn >= _NBUF)
            def _():
                wait_scatter(bn)

            load_and_gather(jn, bn)
        return carry


    n_full = (NCHUNK - PRO - _NBUF) // _NBUF + 1
    lax.fori_loop(0, n_full, body, 0)
    for j in range(_NBUF * n_full, NCHUNK):
        drain_and_scatter(j % _NBUF)
        if j + PRO < NCHUNK:
            bn = (j + PRO) % _NBUF
            wait_scatter(bn)
            load_and_gather(j + PRO, bn)
    for b in range(_NBUF):
        wait_scatter(b)
    plsc.subcore_barrier()
    pltpu.sync_copy(acc.at[pl.ds(s * RPT, RPT)],
                    out_hbm.at[c, pl.ds(s * RPT, RPT)])


@functools.cache
def _make_pool_kernel():
    return functools.partial(
        pl.kernel,
        out_type=[
            jax.ShapeDtypeStruct((NC, S, D), jnp.float32),
            jax.ShapeDtypeStruct((NC, S, D), jnp.float32),
            jax.ShapeDtypeStruct((NC, 16, 128), jnp.float32),
        ],
        mesh=_mesh(),
        scratch_types=[
            pltpu.VMEM((PK,), jnp.int32),
            pltpu.VMEM((PK, D), jnp.float32),
            pltpu.VMEM((PK, D), jnp.float32),
            pltpu.VMEM((PK,), jnp.int32),
            pltpu.VMEM((PK,), jnp.int32),
            pltpu.VMEM((PK, 128), jnp.float32),
            pltpu.VMEM_SHARED((S, D), jnp.float32),
            pltpu.VMEM_SHARED((S, D), jnp.float32),
            pltpu.VMEM_SHARED((16, 128), jnp.float32),
            pltpu.SemaphoreType.DMA,
        ],
    )(_pool_body)


def _pool_body(h1_hbm, h2_hbm, seg_hbm, zrow_hbm, eye_hbm,
               out1_hbm, out2_hbm, outc_hbm,
               seg_v, r1_v, r2_v, rr_v, cc_v, oneh_v, acc1, acc2, accc, sem):
    c = lax.axis_index("c")
    s = lax.axis_index("s")
    t = c * NS + s
    pltpu.sync_copy(zrow_hbm, acc1.at[pl.ds(s * SPT, SPT)])
    pltpu.sync_copy(zrow_hbm, acc2.at[pl.ds(s * SPT, SPT)])

    @pl.when(s < 2)
    def _():
        pltpu.sync_copy(zrow_hbm.at[pl.ds(0, 8)], accc.at[pl.ds(s * 8, 8)])

    plsc.subcore_barrier()

    for k in range(4):
        cid = k * NW + t

        @pl.when(cid < PCH)
        def _():
            base = pl.multiple_of(cid * PK, 8)
            pltpu.sync_copy(seg_hbm.at[pl.ds(base, PK)], seg_v)
            pltpu.sync_copy(h1_hbm.at[pl.ds(base, PK)], r1_v)
            pltpu.sync_copy(h2_hbm.at[pl.ds(base, PK)], r2_v)
            pltpu.sync_copy(r1_v, acc1.at[seg_v], add=True)
            pltpu.sync_copy(r2_v, acc2.at[seg_v], add=True)
            for q in range(PK // 16):
                seg16 = seg_v[pl.ds(16 * q, 16)]
                rr_v[pl.ds(16 * q, 16)] = lax.shift_right_logical(seg16, 7)
                cc_v[pl.ds(16 * q, 16)] = (
                    lax.bitwise_and(seg16, 127) + t * 128)
            pltpu.async_copy(eye_hbm.at[cc_v], oneh_v, sem).wait()
            pltpu.sync_copy(oneh_v, accc.at[rr_v], add=True)

    plsc.subcore_barrier()
    pltpu.sync_copy(acc1.at[pl.ds(s * SPT, SPT)], out1_hbm.at[c, pl.ds(s * SPT, SPT)])
    pltpu.sync_copy(acc2.at[pl.ds(s * SPT, SPT)], out2_hbm.at[c, pl.ds(s * SPT, SPT)])

    @pl.when(s < 2)
    def _():
        pltpu.sync_copy(accc.at[pl.ds(s * 8, 8)], outc_hbm.at[c, pl.ds(s * 8, 8)])


def _degmerge_body(p_ref, out_ref):
    def body(i, acc):
        return acc + p_ref[pl.ds(i * NPR, NPR), :]

    out_ref[...] = lax.fori_loop(
        0, NW, body, jnp.zeros((NPR, 128), jnp.float32))


def _mm1_body(x_ref, w_ref, d_ref, hs_ref, dis_ref):
    deg = d_ref[...] + 1.0
    dis = lax.rsqrt(deg)
    m = jnp.dot(x_ref[...], w_ref[...], preferred_element_type=jnp.float32)
    hs_ref[...] = m * dis
    dis_ref[...] = jnp.broadcast_to(dis, (RB, 16))


def _ep1_body(p0_ref, p1_ref, hs_ref, dis_ref, b_ref, g_ref, stats_ref):
    g = dis_ref[:, 0:1] * (p0_ref[0] + p1_ref[0] + hs_ref[...]) + b_ref[...]
    g_ref[...] = g

    @pl.when(pl.program_id(0) == 0)
    def _():
        stats_ref[...] = jnp.zeros_like(stats_ref)

    upd = jnp.concatenate(
        [jnp.sum(g, axis=0, keepdims=True),
         jnp.sum(g * g, axis=0, keepdims=True),
         jnp.zeros((6, D), jnp.float32)], axis=0)
    stats_ref[...] += upd


def _bn_mm2_body(g_ref, stats_ref, gamma_ref, beta_ref, w_ref, dis_ref,
                 h1_ref, hs2_ref):
    inv_n = 1.0 / N
    mean = stats_ref[0:1, :] * inv_n
    var = stats_ref[1:2, :] * inv_n - mean * mean
    inv = lax.rsqrt(var + 1e-5)
    h1 = (g_ref[...] - mean) * inv * gamma_ref[...] + beta_ref[...]
    h1 = jnp.maximum(h1, 0.0)
    h1_ref[...] = h1
    m = jnp.dot(h1, w_ref[...], preferred_element_type=jnp.float32)
    hs2_ref[...] = m * dis_ref[:, 0:1]


def _ep2_body(p0_ref, p1_ref, hs_ref, dis_ref, b_ref, h2_ref):
    g = dis_ref[:, 0:1] * (p0_ref[0] + p1_ref[0] + hs_ref[...]) + b_ref[...]
    h2_ref[...] = jnp.maximum(g, 0.0)


def _final_body(s1_ref, s2_ref, c0_ref, c1_ref, out_ref):
    cnt = jnp.maximum(c0_ref[0] + c1_ref[0], 1.0)
    out_ref[:, :D] = (s1_ref[0] + s1_ref[1]) / cnt
    out_ref[:, D:] = (s2_ref[0] + s2_ref[1]) / cnt


def _row_spec(shape):
    return pl.BlockSpec(shape, lambda i: (i, 0))


def _fixed_spec(shape):
    return pl.BlockSpec(shape, lambda i: tuple(0 for _ in shape))


def _part_spec(core):
    return pl.BlockSpec((1, RB, D), lambda i, c=core: (c, i, 0))


def kernel(x, sub_edge_index, node_to_subgraph, W1, b1, gamma, beta, W2, b2):
    src = sub_edge_index[0]
    dst = sub_edge_index[1]
    zrows = jnp.zeros((RPT, D), jnp.float32)
    eye = jnp.tile(jnp.eye(128, dtype=jnp.float32), (NW, 1))

    degp = _make_deg_kernel()(dst, eye, zrows)
    degsum = pl.pallas_call(
        _degmerge_body,
        out_shape=jax.ShapeDtypeStruct((NPR, 128), jnp.float32),
    )(degp.reshape(NC * NS * NPR, 128))
    deg1 = degsum.reshape(NPAD, 1)

    hs1, dis = pl.pallas_call(
        _mm1_body,
        grid=(GRID,),
        in_specs=[_row_spec((RB, D)), _fixed_spec((D, D)),
                  pl.BlockSpec((RB, 1), lambda i: (i, 0))],
        out_specs=[_row_spec((RB, D)), _row_spec((RB, 16))],
        out_shape=[jax.ShapeDtypeStruct((N, D), jnp.float32),
                   jax.ShapeDtypeStruct((N, 16), jnp.float32)],
    )(x, W1, deg1)

    parts1 = _make_agg_kernel()(hs1, src, dst, zrows)

    g1, stats = pl.pallas_call(
        _ep1_body,
        grid=(GRID,),
        in_specs=[_part_spec(0), _part_spec(1), _row_spec((RB, D)),
                  _row_spec((RB, 16)), _fixed_spec((1, D))],
        out_specs=[_row_spec((RB, D)), _fixed_spec((8, D))],
        out_shape=[jax.ShapeDtypeStruct((N, D), jnp.float32),
                   jax.ShapeDtypeStruct((8, D), jnp.float32)],
    )(parts1, parts1, hs1, dis, b1.reshape(1, D))

    h1, hs2 = pl.pallas_call(
        _bn_mm2_body,
        grid=(GRID,),
        in_specs=[_row_spec((RB, D)), _fixed_spec((8, D)), _fixed_spec((1, D)),
                  _fixed_spec((1, D)), _fixed_spec((D, D)), _row_spec((RB, 16))],
        out_specs=[_row_spec((RB, D)), _row_spec((RB, D))],
        out_shape=[jax.ShapeDtypeStruct((N, D), jnp.float32),
                   jax.ShapeDtypeStruct((N, D), jnp.float32)],
    )(g1, stats, gamma.reshape(1, D), beta.reshape(1, D), W2, dis)

    parts2 = _make_agg_kernel()(hs2, src, dst, zrows)

    h2 = pl.pallas_call(
        _ep2_body,
        grid=(GRID,),
        in_specs=[_part_spec(0), _part_spec(1), _row_spec((RB, D)),
                  _row_spec((RB, 16)), _fixed_spec((1, D))],
        out_specs=_row_spec((RB, D)),
        out_shape=jax.ShapeDtypeStruct((N, D), jnp.float32),
    )(parts2, parts2, hs2, dis, b2.reshape(1, D))

    psum1, psum2, pcnt = _make_pool_kernel()(
        h1, h2, node_to_subgraph, jnp.zeros((SPT, D), jnp.float32), eye)
    pcnt = pcnt.reshape(NC, 16 * 128, 1)

    out = pl.pallas_call(
        _final_body,
        grid=(1,),
        in_specs=[pl.BlockSpec((NC, S, D), lambda i: (0, 0, 0)),
                  pl.BlockSpec((NC, S, D), lambda i: (0, 0, 0)),
                  pl.BlockSpec((1, S, 1), lambda i: (0, 0, 0)),
                  pl.BlockSpec((1, S, 1), lambda i: (1, 0, 0))],
        out_specs=pl.BlockSpec((S, 2 * D), lambda i: (0, 0)),
        out_shape=jax.ShapeDtypeStruct((S, 2 * D), jnp.float32),
    )(psum1, psum2, pcnt, pcnt)
    return out


# speedup vs baseline: 17.5775x; 1.0361x over previous
import functools

import jax
import jax.numpy as jnp
from jax import lax
from jax.experimental import pallas as pl
from jax.experimental.pallas import tpu as pltpu
from jax.experimental.pallas import tpu_sc as plsc

N = 10000
E = 320000
D = 128
S = 512

NC = 2
NS = 16
NW = NC * NS
ET = E // NW
EK = 80
NCHUNK = ET // EK
NPAD = 10240
RPT = NPAD // NS
RB = 400
GRID = N // RB
PK = 80
PCH = N // PK
SPT = S // NS


def _mesh():
    return plsc.VectorSubcoreMesh(
        core_axis_name="c", subcore_axis_name="s", num_cores=NC, num_subcores=NS)


NPR = NPAD // 128


@functools.cache
def _make_deg_kernel():
    return functools.partial(
        pl.kernel,
        out_type=jax.ShapeDtypeStruct((NC, NS * NPR, 128), jnp.float32),
        mesh=_mesh(),
        scratch_types=(
            [pltpu.VMEM((EK,), jnp.int32)] * (3 * _NBUF)
            + [pltpu.VMEM((EK, 128), jnp.float32)] * _NBUF
            + [pltpu.VMEM_SHARED((NS * NPR, 128), jnp.float32)]
            + [pltpu.SemaphoreType.DMA] * (2 * _NBUF)
        ),
    )(_deg_body)


def _deg_body(dst_hbm, eye_hbm, zeros_hbm, out_hbm, *refs):
    idx_vs = refs[0:_NBUF]
    r_vs = refs[_NBUF:2 * _NBUF]
    c_vs = refs[2 * _NBUF:3 * _NBUF]
    oneh_vs = refs[3 * _NBUF:4 * _NBUF]
    acc = refs[4 * _NBUF]
    sems = refs[4 * _NBUF + 1:5 * _NBUF + 1]
    sc_sems = refs[5 * _NBUF + 1:6 * _NBUF + 1]
    c = lax.axis_index("c")
    s = lax.axis_index("s")
    t = c * NS + s
    pltpu.sync_copy(zeros_hbm.at[pl.ds(0, NPR)], acc.at[pl.ds(s * NPR, NPR)])

    def load_and_gather(j, b):
        base = pl.multiple_of(t * ET + j * EK, 8)
        pltpu.sync_copy(dst_hbm.at[pl.ds(base, EK)], idx_vs[b])
        for k in range(EK // 16):
            idx16 = idx_vs[b][pl.ds(16 * k, 16)]
            r_vs[b][pl.ds(16 * k, 16)] = (
                lax.shift_right_logical(idx16, 7) + s * NPR)
            c_vs[b][pl.ds(16 * k, 16)] = (
                lax.bitwise_and(idx16, 127) + t * 128)
        pltpu.async_copy(eye_hbm.at[c_vs[b]], oneh_vs[b], sems[b])

    def drain_and_scatter(b):
        pltpu.make_async_copy(eye_hbm.at[c_vs[b]], oneh_vs[b], sems[b]).wait()
        pltpu.async_copy(oneh_vs[b], acc.at[r_vs[b]], sc_sems[b], add=True)

    def wait_scatter(b):
        pltpu.make_async_copy(oneh_vs[b], acc.at[r_vs[b]], sc_sems[b]).wait()

    PRO = _PRO
    for j in range(PRO):
        load_and_gather(j, j)

    def body(i, carry):
        j0 = _NBUF * i
        for b in range(_NBUF):
            drain_and_scatter(b)
            bn = (b + PRO) % _NBUF
            jn = j0 + b + PRO

            @pl.when(jn >= _NBUF)
            def _():
                wait_scatter(bn)

            load_and_gather(jn, bn)
        return carry

    n_full = (NCHUNK - PRO - _NBUF) // _NBUF + 1
    lax.fori_loop(0, n_full, body, 0)
    for j in range(_NBUF * n_full, NCHUNK):
        drain_and_scatter(j % _NBUF)
        if j + PRO < NCHUNK:
            bn = (j + PRO) % _NBUF
            wait_scatter(bn)
            load_and_gather(j + PRO, bn)
    for b in range(_NBUF):
        wait_scatter(b)
    pltpu.sync_copy(acc.at[pl.ds(s * NPR, NPR)],
                    out_hbm.at[c, pl.ds(s * NPR, NPR)])


_NBUF = 4
_PRO = 2


@functools.cache
def _make_agg_kernel():
    return functools.partial(
        pl.kernel,
        out_type=jax.ShapeDtypeStruct((NC, NPAD, D), jnp.float32),
        mesh=_mesh(),
        scratch_types=(
            [pltpu.VMEM((EK,), jnp.int32)] * (2 * _NBUF)
            + [pltpu.VMEM((EK, D), jnp.float32)] * _NBUF
            + [pltpu.VMEM_SHARED((NPAD, D), jnp.float32)]
            + [pltpu.SemaphoreType.DMA] * (2 * _NBUF)
        ),
    )(_agg_body)


def _agg_body(hs_hbm, src_hbm, dst_hbm, zeros_hbm, out_hbm, *refs):
    src_vs = refs[0:_NBUF]
    dst_vs = refs[_NBUF:2 * _NBUF]
    rows_vs = refs[2 * _NBUF:3 * _NBUF]
    acc = refs[3 * _NBUF]
    sems = refs[3 * _NBUF + 1:4 * _NBUF + 1]
    sc_sems = refs[4 * _NBUF + 1:5 * _NBUF + 1]
    c = lax.axis_index("c")
    s = lax.axis_index("s")
    t = c * NS + s
    pltpu.sync_copy(zeros_hbm, acc.at[pl.ds(s * RPT, RPT)])
    plsc.subcore_barrier()

    def load_and_gather(j, b):
        base = pl.multiple_of(t * ET + j * EK, 8)
        pltpu.sync_copy(src_hbm.at[pl.ds(base, EK)], src_vs[b])
        pltpu.sync_copy(dst_hbm.at[pl.ds(base, EK)], dst_vs[b])
        pltpu.async_copy(hs_hbm.at[src_vs[b]], rows_vs[b], sems[b])

    def drain_and_scatter(b):
        pltpu.make_async_copy(hs_hbm.at[src_vs[b]], rows_vs[b], sems[b]).wait()
        pltpu.async_copy(rows_vs[b], acc.at[dst_vs[b]], sc_sems[b], add=True)

    def wait_scatter(b):
        pltpu.make_async_copy(rows_vs[b], acc.at[dst_vs[b]], sc_sems[b]).wait()

    PRO = _PRO
    for j in range(PRO):
        load_and_gather(j, j)

    def body(i, carry):
        j0 = _NBUF * i
        for b in range(_NBUF):
            drain_and_scatter(b)
            bn = (b + PRO) % _NBUF
            jn = j0 + b + PRO

            @pl.when(jn >= _NBUF)
            def _():
                wait_scatter(bn)

            load_and_gather(jn, bn)
        return carry


    n_full = (NCHUNK - PRO - _NBUF) // _NBUF + 1
    lax.fori_loop(0, n_full, body, 0)
    for j in range(_NBUF * n_full, NCHUNK):
        drain_and_scatter(j % _NBUF)
        if j + PRO < NCHUNK:
            bn = (j + PRO) % _NBUF
            wait_scatter(bn)
            load_and_gather(j + PRO, bn)
    for b in range(_NBUF):
        wait_scatter(b)
    plsc.subcore_barrier()
    pltpu.sync_copy(acc.at[pl.ds(s * RPT, RPT)],
                    out_hbm.at[c, pl.ds(s * RPT, RPT)])


@functools.cache
def _make_pool_kernel():
    return functools.partial(
        pl.kernel,
        out_type=[
            jax.ShapeDtypeStruct((NC, S, D), jnp.float32),
            jax.ShapeDtypeStruct((NC, S, D), jnp.float32),
            jax.ShapeDtypeStruct((NC, 16, 128), jnp.float32),
        ],
        mesh=_mesh(),
        scratch_types=[
            pltpu.VMEM((PK,), jnp.int32),
            pltpu.VMEM((PK, D), jnp.float32),
            pltpu.VMEM((PK, D), jnp.float32),
            pltpu.VMEM((PK,), jnp.int32),
            pltpu.VMEM((PK,), jnp.int32),
            pltpu.VMEM((PK, 128), jnp.float32),
            pltpu.VMEM_SHARED((S, D), jnp.float32),
            pltpu.VMEM_SHARED((S, D), jnp.float32),
            pltpu.VMEM_SHARED((16, 128), jnp.float32),
            pltpu.SemaphoreType.DMA,
        ],
    )(_pool_body)


def _pool_body(h1_hbm, h2_hbm, seg_hbm, zrow_hbm, eye_hbm,
               out1_hbm, out2_hbm, outc_hbm,
               seg_v, r1_v, r2_v, rr_v, cc_v, oneh_v, acc1, acc2, accc, sem):
    c = lax.axis_index("c")
    s = lax.axis_index("s")
    t = c * NS + s
    pltpu.sync_copy(zrow_hbm, acc1.at[pl.ds(s * SPT, SPT)])
    pltpu.sync_copy(zrow_hbm, acc2.at[pl.ds(s * SPT, SPT)])

    @pl.when(s < 2)
    def _():
        pltpu.sync_copy(zrow_hbm.at[pl.ds(0, 8)], accc.at[pl.ds(s * 8, 8)])

    plsc.subcore_barrier()

    for k in range(4):
        cid = k * NW + t

        @pl.when(cid < PCH)
        def _():
            base = pl.multiple_of(cid * PK, 8)
            pltpu.sync_copy(seg_hbm.at[pl.ds(base, PK)], seg_v)
            pltpu.sync_copy(h1_hbm.at[pl.ds(base, PK)], r1_v)
            pltpu.sync_copy(h2_hbm.at[pl.ds(base, PK)], r2_v)
            pltpu.sync_copy(r1_v, acc1.at[seg_v], add=True)
            pltpu.sync_copy(r2_v, acc2.at[seg_v], add=True)
            for q in range(PK // 16):
                seg16 = seg_v[pl.ds(16 * q, 16)]
                rr_v[pl.ds(16 * q, 16)] = lax.shift_right_logical(seg16, 7)
                cc_v[pl.ds(16 * q, 16)] = (
                    lax.bitwise_and(seg16, 127) + t * 128)
            pltpu.async_copy(eye_hbm.at[cc_v], oneh_v, sem).wait()
            pltpu.sync_copy(oneh_v, accc.at[rr_v], add=True)

    plsc.subcore_barrier()
    pltpu.sync_copy(acc1.at[pl.ds(s * SPT, SPT)], out1_hbm.at[c, pl.ds(s * SPT, SPT)])
    pltpu.sync_copy(acc2.at[pl.ds(s * SPT, SPT)], out2_hbm.at[c, pl.ds(s * SPT, SPT)])

    @pl.when(s < 2)
    def _():
        pltpu.sync_copy(accc.at[pl.ds(s * 8, 8)], outc_hbm.at[c, pl.ds(s * 8, 8)])


def _degmerge_body(p_ref, out_ref):
    def body(i, acc):
        return acc + p_ref[pl.ds(i * NPR, NPR), :]

    out_ref[...] = lax.fori_loop(
        0, NW, body, jnp.zeros((NPR, 128), jnp.float32))


def _mm1_body(x_ref, w_ref, d_ref, hs_ref, dis_ref):
    deg = d_ref[...] + 1.0
    dis = lax.rsqrt(deg)
    m = jnp.dot(x_ref[...], w_ref[...], preferred_element_type=jnp.float32)
    hs_ref[...] = m * dis
    dis_ref[...] = jnp.broadcast_to(dis, (RB, 16))


def _ep1_body(p0_ref, p1_ref, hs_ref, dis_ref, b_ref, g_ref, stats_ref):
    g = dis_ref[:, 0:1] * (p0_ref[0] + p1_ref[0] + hs_ref[...]) + b_ref[...]
    g_ref[...] = g

    @pl.when(pl.program_id(0) == 0)
    def _():
        stats_ref[...] = jnp.zeros_like(stats_ref)

    upd = jnp.concatenate(
        [jnp.sum(g, axis=0, keepdims=True),
         jnp.sum(g * g, axis=0, keepdims=True),
         jnp.zeros((6, D), jnp.float32)], axis=0)
    stats_ref[...] += upd


def _bn_mm2_body(g_ref, stats_ref, gamma_ref, beta_ref, w_ref, dis_ref,
                 h1_ref, hs2_ref):
    inv_n = 1.0 / N
    mean = stats_ref[0:1, :] * inv_n
    var = stats_ref[1:2, :] * inv_n - mean * mean
    inv = lax.rsqrt(var + 1e-5)
    h1 = (g_ref[...] - mean) * inv * gamma_ref[...] + beta_ref[...]
    h1 = jnp.maximum(h1, 0.0)
    h1_ref[...] = h1
    m = jnp.dot(h1, w_ref[...], preferred_element_type=jnp.float32)
    hs2_ref[...] = m * dis_ref[:, 0:1]


def _ep2_body(p0_ref, p1_ref, hs_ref, dis_ref, b_ref, h2_ref):
    g = dis_ref[:, 0:1] * (p0_ref[0] + p1_ref[0] + hs_ref[...]) + b_ref[...]
    h2_ref[...] = jnp.maximum(g, 0.0)


def _final_body(s1_ref, s2_ref, c0_ref, c1_ref, out_ref):
    cnt = jnp.maximum(c0_ref[0] + c1_ref[0], 1.0)
    out_ref[:, :D] = (s1_ref[0] + s1_ref[1]) / cnt
    out_ref[:, D:] = (s2_ref[0] + s2_ref[1]) / cnt


def _row_spec(shape):
    return pl.BlockSpec(shape, lambda i: (i, 0))


def _fixed_spec(shape):
    return pl.BlockSpec(shape, lambda i: tuple(0 for _ in shape))


def _part_spec(core):
    return pl.BlockSpec((1, RB, D), lambda i, c=core: (c, i, 0))


def kernel(x, sub_edge_index, node_to_subgraph, W1, b1, gamma, beta, W2, b2):
    src = sub_edge_index[0]
    dst = sub_edge_index[1]
    zrows = jnp.zeros((RPT, D), jnp.float32)
    eye = jnp.tile(jnp.eye(128, dtype=jnp.float32), (NW, 1))

    degp = _make_deg_kernel()(dst, eye, zrows)
    degsum = pl.pallas_call(
        _degmerge_body,
        out_shape=jax.ShapeDtypeStruct((NPR, 128), jnp.float32),
    )(degp.reshape(NC * NS * NPR, 128))
    deg1 = degsum.reshape(NPAD, 1)

    hs1, dis = pl.pallas_call(
        _mm1_body,
        grid=(GRID,),
        in_specs=[_row_spec((RB, D)), _fixed_spec((D, D)),
                  pl.BlockSpec((RB, 1), lambda i: (i, 0))],
        out_specs=[_row_spec((RB, D)), _row_spec((RB, 16))],
        out_shape=[jax.ShapeDtypeStruct((N, D), jnp.float32),
                   jax.ShapeDtypeStruct((N, 16), jnp.float32)],
    )(x, W1, deg1)

    parts1 = _make_agg_kernel()(hs1, src, dst, zrows)

    g1, stats = pl.pallas_call(
        _ep1_body,
        grid=(GRID,),
        in_specs=[_part_spec(0), _part_spec(1), _row_spec((RB, D)),
                  _row_spec((RB, 16)), _fixed_spec((1, D))],
        out_specs=[_row_spec((RB, D)), _fixed_spec((8, D))],
        out_shape=[jax.ShapeDtypeStruct((N, D), jnp.float32),
                   jax.ShapeDtypeStruct((8, D), jnp.float32)],
    )(parts1, parts1, hs1, dis, b1.reshape(1, D))

    h1, hs2 = pl.pallas_call(
        _bn_mm2_body,
        grid=(GRID,),
        in_specs=[_row_spec((RB, D)), _fixed_spec((8, D)), _fixed_spec((1, D)),
                  _fixed_spec((1, D)), _fixed_spec((D, D)), _row_spec((RB, 16))],
        out_specs=[_row_spec((RB, D)), _row_spec((RB, D))],
        out_shape=[jax.ShapeDtypeStruct((N, D), jnp.float32),
                   jax.ShapeDtypeStruct((N, D), jnp.float32)],
    )(g1, stats, gamma.reshape(1, D), beta.reshape(1, D), W2, dis)

    parts2 = _make_agg_kernel()(hs2, src, dst, zrows)

    h2 = pl.pallas_call(
        _ep2_body,
        grid=(GRID,),
        in_specs=[_part_spec(0), _part_spec(1), _row_spec((RB, D)),
                  _row_spec((RB, 16)), _fixed_spec((1, D))],
        out_specs=_row_spec((RB, D)),
        out_shape=jax.ShapeDtypeStruct((N, D), jnp.float32),
    )(parts2, parts2, hs2, dis, b2.reshape(1, D))

    psum1, psum2, pcnt = _make_pool_kernel()(
        h1, h2, node_to_subgraph, jnp.zeros((SPT, D), jnp.float32), eye)
    pcnt = pcnt.reshape(NC, 16 * 128, 1)

    out = pl.pallas_call(
        _final_body,
        grid=(1,),
        in_specs=[pl.BlockSpec((NC, S, D), lambda i: (0, 0, 0)),
                  pl.BlockSpec((NC, S, D), lambda i: (0, 0, 0)),
                  pl.BlockSpec((1, S, 1), lambda i: (0, 0, 0)),
                  pl.BlockSpec((1, S, 1), lambda i: (1, 0, 0))],
        out_specs=pl.BlockSpec((S, 2 * D), lambda i: (0, 0)),
        out_shape=jax.ShapeDtypeStruct((S, 2 * D), jnp.float32),
    )(psum1, psum2, pcnt, pcnt)
    return out


# speedup vs baseline: 18.6129x; 1.0589x over previous
import functools

import jax
import jax.numpy as jnp
from jax import lax
from jax.experimental import pallas as pl
from jax.experimental.pallas import tpu as pltpu
from jax.experimental.pallas import tpu_sc as plsc

N = 10000
E = 320000
D = 128
S = 512

NC = 2
NS = 16
NW = NC * NS
ET = E // NW
EK = 80
NCHUNK = ET // EK
NPAD = 10240
RPT = NPAD // NS
RB = 400
GRID = N // RB
PK = 80
PCH = N // PK
SPT = S // NS


def _mesh():
    return plsc.VectorSubcoreMesh(
        core_axis_name="c", subcore_axis_name="s", num_cores=NC, num_subcores=NS)


NPR = NPAD // 128


@functools.cache
def _make_deg_kernel():
    return functools.partial(
        pl.kernel,
        out_type=jax.ShapeDtypeStruct((NC, NS * NPR, 128), jnp.float32),
        mesh=_mesh(),
        scratch_types=(
            [pltpu.VMEM((EK,), jnp.int32)] * (3 * _NBUF)
            + [pltpu.VMEM((EK, 128), jnp.float32)] * _NBUF
            + [pltpu.VMEM_SHARED((NS * NPR, 128), jnp.float32)]
            + [pltpu.SemaphoreType.DMA] * (2 * _NBUF)
        ),
    )(_deg_body)


def _deg_body(dst_hbm, eye_hbm, zeros_hbm, out_hbm, *refs):
    idx_vs = refs[0:_NBUF]
    r_vs = refs[_NBUF:2 * _NBUF]
    c_vs = refs[2 * _NBUF:3 * _NBUF]
    oneh_vs = refs[3 * _NBUF:4 * _NBUF]
    acc = refs[4 * _NBUF]
    sems = refs[4 * _NBUF + 1:5 * _NBUF + 1]
    sc_sems = refs[5 * _NBUF + 1:6 * _NBUF + 1]
    c = lax.axis_index("c")
    s = lax.axis_index("s")
    t = c * NS + s
    pltpu.sync_copy(zeros_hbm.at[pl.ds(0, NPR)], acc.at[pl.ds(s * NPR, NPR)])

    def load_and_gather(j, b):
        base = pl.multiple_of(t * ET + j * EK, 8)
        pltpu.sync_copy(dst_hbm.at[pl.ds(base, EK)], idx_vs[b])
        for k in range(EK // 16):
            idx16 = idx_vs[b][pl.ds(16 * k, 16)]
            r_vs[b][pl.ds(16 * k, 16)] = (
                lax.shift_right_logical(idx16, 7) + s * NPR)
            c_vs[b][pl.ds(16 * k, 16)] = (
                lax.bitwise_and(idx16, 127) + t * 128)
        pltpu.async_copy(eye_hbm.at[c_vs[b]], oneh_vs[b], sems[b])

    def drain_and_scatter(b):
        pltpu.make_async_copy(eye_hbm.at[c_vs[b]], oneh_vs[b], sems[b]).wait()
        pltpu.async_copy(oneh_vs[b], acc.at[r_vs[b]], sc_sems[b], add=True)

    def wait_scatter(b):
        pltpu.make_async_copy(oneh_vs[b], acc.at[r_vs[b]], sc_sems[b]).wait()

    PRO = _PRO
    for j in range(PRO):
        load_and_gather(j, j)

    def body(i, carry):
        j0 = _NBUF * i
        for b in range(_NBUF):
            drain_and_scatter(b)
            bn = (b + PRO) % _NBUF
            jn = j0 + b + PRO

            @pl.when(jn >= _NBUF)
            def _():
                wait_scatter(bn)

            load_and_gather(jn, bn)
        return carry

    n_full = (NCHUNK - PRO - _NBUF) // _NBUF + 1
    lax.fori_loop(0, n_full, body, 0)
    for j in range(_NBUF * n_full, NCHUNK):
        drain_and_scatter(j % _NBUF)
        if j + PRO < NCHUNK:
            bn = (j + PRO) % _NBUF
            wait_scatter(bn)
            load_and_gather(j + PRO, bn)
    for b in range(_NBUF):
        wait_scatter(b)
    pltpu.sync_copy(acc.at[pl.ds(s * NPR, NPR)],
                    out_hbm.at[c, pl.ds(s * NPR, NPR)])


_NBUF = 4
_PRO = 2
_GK = 8


@functools.cache
def _make_agg_kernel():
    return functools.partial(
        pl.kernel,
        out_type=jax.ShapeDtypeStruct((NC, NPAD, D), jnp.float32),
        mesh=_mesh(),
        scratch_types=(
            [pltpu.VMEM((EK,), jnp.int32)] * (2 * _NBUF)
            + [pltpu.VMEM((_GK * EK,), jnp.int32)] * 2
            + [pltpu.VMEM((EK, D), jnp.float32)] * _NBUF
            + [pltpu.VMEM_SHARED((NPAD, D), jnp.float32)]
            + [pltpu.SemaphoreType.DMA] * (2 * _NBUF)
        ),
    )(_agg_body)


def _agg_body(hs_hbm, src_hbm, dst_hbm, zeros_hbm, out_hbm, *refs):
    src_vs = refs[0:_NBUF]
    dst_vs = refs[_NBUF:2 * _NBUF]
    sb, db = refs[2 * _NBUF:2 * _NBUF + 2]
    rows_vs = refs[2 * _NBUF + 2:3 * _NBUF + 2]
    acc = refs[3 * _NBUF + 2]
    sems = refs[3 * _NBUF + 3:4 * _NBUF + 3]
    sc_sems = refs[4 * _NBUF + 3:5 * _NBUF + 3]
    c = lax.axis_index("c")
    s = lax.axis_index("s")
    t = c * NS + s
    pltpu.sync_copy(zeros_hbm, acc.at[pl.ds(s * RPT, RPT)])
    plsc.subcore_barrier()

    def load_and_gather(j, b):
        @pl.when(j % _GK == 0)
        def _():
            base = pl.multiple_of(t * ET + j * EK, 8)
            pltpu.sync_copy(src_hbm.at[pl.ds(base, _GK * EK)], sb)
            pltpu.sync_copy(dst_hbm.at[pl.ds(base, _GK * EK)], db)

        o = (j % _GK) * EK
        for k in range(EK // 16):
            src_vs[b][pl.ds(16 * k, 16)] = sb[pl.ds(o + 16 * k, 16)]
            dst_vs[b][pl.ds(16 * k, 16)] = db[pl.ds(o + 16 * k, 16)]
        pltpu.async_copy(hs_hbm.at[src_vs[b]], rows_vs[b], sems[b])

    def drain_and_scatter(b):
        pltpu.make_async_copy(hs_hbm.at[src_vs[b]], rows_vs[b], sems[b]).wait()
        pltpu.async_copy(rows_vs[b], acc.at[dst_vs[b]], sc_sems[b], add=True)

    def wait_scatter(b):
        pltpu.make_async_copy(rows_vs[b], acc.at[dst_vs[b]], sc_sems[b]).wait()

    PRO = _PRO
    for j in range(PRO):
        load_and_gather(j, j)

    def body(i, carry):
        j0 = _NBUF * i
        for b in range(_NBUF):
            drain_and_scatter(b)
            bn = (b + PRO) % _NBUF
            jn = j0 + b + PRO

            @pl.when(jn >= _NBUF)
            def _():
                wait_scatter(bn)

            load_and_gather(jn, bn)
        return carry


    n_full = (NCHUNK - PRO - _NBUF) // _NBUF + 1
    lax.fori_loop(0, n_full, body, 0)
    for j in range(_NBUF * n_full, NCHUNK):
        drain_and_scatter(j % _NBUF)
        if j + PRO < NCHUNK:
            bn = (j + PRO) % _NBUF
            wait_scatter(bn)
            load_and_gather(j + PRO, bn)
    for b in range(_NBUF):
        wait_scatter(b)
    plsc.subcore_barrier()
    pltpu.sync_copy(acc.at[pl.ds(s * RPT, RPT)],
                    out_hbm.at[c, pl.ds(s * RPT, RPT)])


@functools.cache
def _make_pool_kernel():
    return functools.partial(
        pl.kernel,
        out_type=[
            jax.ShapeDtypeStruct((NC, S, D), jnp.float32),
            jax.ShapeDtypeStruct((NC, S, D), jnp.float32),
            jax.ShapeDtypeStruct((NC, 16, 128), jnp.float32),
        ],
        mesh=_mesh(),
        scratch_types=[
            pltpu.VMEM((PK,), jnp.int32),
            pltpu.VMEM((PK, D), jnp.float32),
            pltpu.VMEM((PK, D), jnp.float32),
            pltpu.VMEM((PK,), jnp.int32),
            pltpu.VMEM((PK,), jnp.int32),
            pltpu.VMEM((PK, 128), jnp.float32),
            pltpu.VMEM_SHARED((S, D), jnp.float32),
            pltpu.VMEM_SHARED((S, D), jnp.float32),
            pltpu.VMEM_SHARED((16, 128), jnp.float32),
            pltpu.SemaphoreType.DMA,
        ],
    )(_pool_body)


def _pool_body(h1_hbm, h2_hbm, seg_hbm, zrow_hbm, eye_hbm,
               out1_hbm, out2_hbm, outc_hbm,
               seg_v, r1_v, r2_v, rr_v, cc_v, oneh_v, acc1, acc2, accc, sem):
    c = lax.axis_index("c")
    s = lax.axis_index("s")
    t = c * NS + s
    pltpu.sync_copy(zrow_hbm, acc1.at[pl.ds(s * SPT, SPT)])
    pltpu.sync_copy(zrow_hbm, acc2.at[pl.ds(s * SPT, SPT)])

    @pl.when(s < 2)
    def _():
        pltpu.sync_copy(zrow_hbm.at[pl.ds(0, 8)], accc.at[pl.ds(s * 8, 8)])

    plsc.subcore_barrier()

    for k in range(4):
        cid = k * NW + t

        @pl.when(cid < PCH)
        def _():
            base = pl.multiple_of(cid * PK, 8)
            pltpu.sync_copy(seg_hbm.at[pl.ds(base, PK)], seg_v)
            pltpu.sync_copy(h1_hbm.at[pl.ds(base, PK)], r1_v)
            pltpu.sync_copy(h2_hbm.at[pl.ds(base, PK)], r2_v)
            pltpu.sync_copy(r1_v, acc1.at[seg_v], add=True)
            pltpu.sync_copy(r2_v, acc2.at[seg_v], add=True)
            for q in range(PK // 16):
                seg16 = seg_v[pl.ds(16 * q, 16)]
                rr_v[pl.ds(16 * q, 16)] = lax.shift_right_logical(seg16, 7)
                cc_v[pl.ds(16 * q, 16)] = (
                    lax.bitwise_and(seg16, 127) + t * 128)
            pltpu.async_copy(eye_hbm.at[cc_v], oneh_v, sem).wait()
            pltpu.sync_copy(oneh_v, accc.at[rr_v], add=True)

    plsc.subcore_barrier()
    pltpu.sync_copy(acc1.at[pl.ds(s * SPT, SPT)], out1_hbm.at[c, pl.ds(s * SPT, SPT)])
    pltpu.sync_copy(acc2.at[pl.ds(s * SPT, SPT)], out2_hbm.at[c, pl.ds(s * SPT, SPT)])

    @pl.when(s < 2)
    def _():
        pltpu.sync_copy(accc.at[pl.ds(s * 8, 8)], outc_hbm.at[c, pl.ds(s * 8, 8)])


def _degmerge_body(p_ref, out_ref):
    def body(i, acc):
        return acc + p_ref[pl.ds(i * NPR, NPR), :]

    out_ref[...] = lax.fori_loop(
        0, NW, body, jnp.zeros((NPR, 128), jnp.float32))


def _mm1_body(x_ref, w_ref, d_ref, hs_ref, dis_ref):
    deg = d_ref[...] + 1.0
    dis = lax.rsqrt(deg)
    m = jnp.dot(x_ref[...], w_ref[...], preferred_element_type=jnp.float32)
    hs_ref[...] = m * dis
    dis_ref[...] = jnp.broadcast_to(dis, (RB, 16))


def _ep1_body(p0_ref, p1_ref, hs_ref, dis_ref, b_ref, g_ref, stats_ref):
    g = dis_ref[:, 0:1] * (p0_ref[0] + p1_ref[0] + hs_ref[...]) + b_ref[...]
    g_ref[...] = g

    @pl.when(pl.program_id(0) == 0)
    def _():
        stats_ref[...] = jnp.zeros_like(stats_ref)

    upd = jnp.concatenate(
        [jnp.sum(g, axis=0, keepdims=True),
         jnp.sum(g * g, axis=0, keepdims=True),
         jnp.zeros((6, D), jnp.float32)], axis=0)
    stats_ref[...] += upd


def _bn_mm2_body(g_ref, stats_ref, gamma_ref, beta_ref, w_ref, dis_ref,
                 h1_ref, hs2_ref):
    inv_n = 1.0 / N
    mean = stats_ref[0:1, :] * inv_n
    var = stats_ref[1:2, :] * inv_n - mean * mean
    inv = lax.rsqrt(var + 1e-5)
    h1 = (g_ref[...] - mean) * inv * gamma_ref[...] + beta_ref[...]
    h1 = jnp.maximum(h1, 0.0)
    h1_ref[...] = h1
    m = jnp.dot(h1, w_ref[...], preferred_element_type=jnp.float32)
    hs2_ref[...] = m * dis_ref[:, 0:1]


def _ep2_body(p0_ref, p1_ref, hs_ref, dis_ref, b_ref, h2_ref):
    g = dis_ref[:, 0:1] * (p0_ref[0] + p1_ref[0] + hs_ref[...]) + b_ref[...]
    h2_ref[...] = jnp.maximum(g, 0.0)


def _final_body(s1_ref, s2_ref, c0_ref, c1_ref, out_ref):
    cnt = jnp.maximum(c0_ref[0] + c1_ref[0], 1.0)
    out_ref[:, :D] = (s1_ref[0] + s1_ref[1]) / cnt
    out_ref[:, D:] = (s2_ref[0] + s2_ref[1]) / cnt


def _row_spec(shape):
    return pl.BlockSpec(shape, lambda i: (i, 0))


def _fixed_spec(shape):
    return pl.BlockSpec(shape, lambda i: tuple(0 for _ in shape))


def _part_spec(core):
    return pl.BlockSpec((1, RB, D), lambda i, c=core: (c, i, 0))


def kernel(x, sub_edge_index, node_to_subgraph, W1, b1, gamma, beta, W2, b2):
    src = jnp.pad(sub_edge_index[0], (0, _GK * EK))
    dst = jnp.pad(sub_edge_index[1], (0, _GK * EK))
    zrows = jnp.zeros((RPT, D), jnp.float32)
    eye = jnp.tile(jnp.eye(128, dtype=jnp.float32), (NW, 1))

    degp = _make_deg_kernel()(dst, eye, zrows)
    degsum = pl.pallas_call(
        _degmerge_body,
        out_shape=jax.ShapeDtypeStruct((NPR, 128), jnp.float32),
    )(degp.reshape(NC * NS * NPR, 128))
    deg1 = degsum.reshape(NPAD, 1)

    hs1, dis = pl.pallas_call(
        _mm1_body,
        grid=(GRID,),
        in_specs=[_row_spec((RB, D)), _fixed_spec((D, D)),
                  pl.BlockSpec((RB, 1), lambda i: (i, 0))],
        out_specs=[_row_spec((RB, D)), _row_spec((RB, 16))],
        out_shape=[jax.ShapeDtypeStruct((N, D), jnp.float32),
                   jax.ShapeDtypeStruct((N, 16), jnp.float32)],
    )(x, W1, deg1)

    parts1 = _make_agg_kernel()(hs1, src, dst, zrows)

    g1, stats = pl.pallas_call(
        _ep1_body,
        grid=(GRID,),
        in_specs=[_part_spec(0), _part_spec(1), _row_spec((RB, D)),
                  _row_spec((RB, 16)), _fixed_spec((1, D))],
        out_specs=[_row_spec((RB, D)), _fixed_spec((8, D))],
        out_shape=[jax.ShapeDtypeStruct((N, D), jnp.float32),
                   jax.ShapeDtypeStruct((8, D), jnp.float32)],
    )(parts1, parts1, hs1, dis, b1.reshape(1, D))

    h1, hs2 = pl.pallas_call(
        _bn_mm2_body,
        grid=(GRID,),
        in_specs=[_row_spec((RB, D)), _fixed_spec((8, D)), _fixed_spec((1, D)),
                  _fixed_spec((1, D)), _fixed_spec((D, D)), _row_spec((RB, 16))],
        out_specs=[_row_spec((RB, D)), _row_spec((RB, D))],
        out_shape=[jax.ShapeDtypeStruct((N, D), jnp.float32),
                   jax.ShapeDtypeStruct((N, D), jnp.float32)],
    )(g1, stats, gamma.reshape(1, D), beta.reshape(1, D), W2, dis)

    parts2 = _make_agg_kernel()(hs2, src, dst, zrows)

    h2 = pl.pallas_call(
        _ep2_body,
        grid=(GRID,),
        in_specs=[_part_spec(0), _part_spec(1), _row_spec((RB, D)),
                  _row_spec((RB, 16)), _fixed_spec((1, D))],
        out_specs=_row_spec((RB, D)),
        out_shape=jax.ShapeDtypeStruct((N, D), jnp.float32),
    )(parts2, parts2, hs2, dis, b2.reshape(1, D))

    psum1, psum2, pcnt = _make_pool_kernel()(
        h1, h2, node_to_subgraph, jnp.zeros((SPT, D), jnp.float32), eye)
    pcnt = pcnt.reshape(NC, 16 * 128, 1)

    out = pl.pallas_call(
        _final_body,
        grid=(1,),
        in_specs=[pl.BlockSpec((NC, S, D), lambda i: (0, 0, 0)),
                  pl.BlockSpec((NC, S, D), lambda i: (0, 0, 0)),
                  pl.BlockSpec((1, S, 1), lambda i: (0, 0, 0)),
                  pl.BlockSpec((1, S, 1), lambda i: (1, 0, 0))],
        out_specs=pl.BlockSpec((S, 2 * D), lambda i: (0, 0)),
        out_shape=jax.ShapeDtypeStruct((S, 2 * D), jnp.float32),
    )(psum1, psum2, pcnt, pcnt)
    return out
